# speedup vs baseline: 1.0742x; 1.0742x over previous
import jax
import jax.numpy as jnp
from jax.experimental import pallas as pl

N = 10000
E = 320000
IN_CH = 128
HID = 64
HEADS = 8
OUT_CH = 1

ROWS = 400


def _node1_body(x_ref, w_ref, acat_ref, h_ref, asd_ref):
    h = jnp.dot(x_ref[...], w_ref[...], preferred_element_type=jnp.float32)
    h_ref[...] = h
    asd_ref[...] = jnp.dot(h, acat_ref[...], preferred_element_type=jnp.float32)


def _node1(x, W1, Acat):
    n = x.shape[0]
    grid = (n // ROWS,)
    return pl.pallas_call(
        _node1_body,
        grid=grid,
        in_specs=[
            pl.BlockSpec((ROWS, IN_CH), lambda i: (i, 0)),
            pl.BlockSpec((IN_CH, HEADS * HID), lambda i: (0, 0)),
            pl.BlockSpec((HEADS * HID, 16), lambda i: (0, 0)),
        ],
        out_specs=[
            pl.BlockSpec((ROWS, HEADS * HID), lambda i: (i, 0)),
            pl.BlockSpec((ROWS, 16), lambda i: (i, 0)),
        ],
        out_shape=[
            jax.ShapeDtypeStruct((n, HEADS * HID), jnp.float32),
            jax.ShapeDtypeStruct((n, 16), jnp.float32),
        ],
    )(x, W1, Acat)


def kernel(x, edge_index, W1, att_src1, att_dst1, b1, W2, att_src2, att_dst2, b2):
    n = N
    loop = jnp.arange(n, dtype=edge_index.dtype)
    src = jnp.concatenate([edge_index[0], loop])
    dst = jnp.concatenate([edge_index[1], loop])

    A_src = jnp.zeros((HEADS * HID, HEADS), jnp.float32)
    A_src = A_src.at[jnp.arange(HEADS * HID), jnp.arange(HEADS * HID) // HID].set(
        att_src1.reshape(-1))
    A_dst = jnp.zeros((HEADS * HID, HEADS), jnp.float32)
    A_dst = A_dst.at[jnp.arange(HEADS * HID), jnp.arange(HEADS * HID) // HID].set(
        att_dst1.reshape(-1))
    Acat = jnp.concatenate([A_src, A_dst], axis=1)

    h, asd = _node1(x, W1, Acat)
    shift = jnp.max(asd[:, :HEADS], axis=0) + jnp.max(asd[:, HEADS:], axis=0)

    a_src = asd[:, :HEADS]
    a_dst = asd[:, HEADS:]

    alpha = a_src[src] + a_dst[dst]
    alpha = jax.nn.leaky_relu(alpha, negative_slope=0.2)
    ealpha = jnp.exp(alpha - shift[None, :])
    seg_sum = jax.ops.segment_sum(ealpha, dst, num_segments=n)
    w = ealpha / (seg_sum[dst] + 1e-16 * jnp.exp(-shift)[None, :])
    hh = h.reshape(n, HEADS, HID)
    msg = hh[src] * w[:, :, None]
    out = jax.ops.segment_sum(msg, dst, num_segments=n).reshape(n, HEADS * HID)
    out = out + b1

    z = jax.nn.elu(out)
    h2 = (z @ W2)
    a_src2 = (h2 * att_src2[0]).sum(-1)
    a_dst2 = (h2 * att_dst2[0]).sum(-1)
    shift2 = jnp.max(a_src2) + jnp.max(a_dst2)
    alpha2 = a_src2[src] + a_dst2[dst]
    alpha2 = jax.nn.leaky_relu(alpha2, negative_slope=0.2)
    e2 = jnp.exp(alpha2 - shift2)
    seg2 = jax.ops.segment_sum(e2, dst, num_segments=n)
    w2 = e2 / (seg2[dst] + 1e-16 * jnp.exp(-shift2))
    out2 = jax.ops.segment_sum(w2 * h2[src, 0], dst, num_segments=n)
    out2 = out2 + b2[0]
    return jax.nn.sigmoid(out2)


# speedup vs baseline: 1.1099x; 1.0333x over previous
import functools

import jax
import jax.numpy as jnp
from jax import lax
from jax.experimental import pallas as pl
from jax.experimental.pallas import tpu as pltpu
from jax.experimental.pallas import tpu_sc as plsc

N = 10000
E = 320000
IN_CH = 128
HID = 64
HEADS = 8
OUT_CH = 1

NP = 10240
NE = E + N
EPT = 10496
EP = 32 * EPT
K = 128
NCHUNK = EPT // K
RPS = NP // 16

ROWS = 400


def _node1_body(x_ref, w_ref, acat_ref, h_ref, asd_ref):
    h = jnp.dot(x_ref[...], w_ref[...], preferred_element_type=jnp.float32)
    h_ref[...] = h
    asd_ref[...] = jnp.dot(h, acat_ref[...], preferred_element_type=jnp.float32)


def _node1(x, W1, Acat):
    n = x.shape[0]
    return pl.pallas_call(
        _node1_body,
        grid=(n // ROWS,),
        in_specs=[
            pl.BlockSpec((ROWS, IN_CH), lambda i: (i, 0)),
            pl.BlockSpec((IN_CH, HEADS * HID), lambda i: (0, 0)),
            pl.BlockSpec((HEADS * HID, 16), lambda i: (0, 0)),
        ],
        out_specs=[
            pl.BlockSpec((ROWS, HEADS * HID), lambda i: (i, 0)),
            pl.BlockSpec((ROWS, 16), lambda i: (i, 0)),
        ],
        out_shape=[
            jax.ShapeDtypeStruct((n, HEADS * HID), jnp.float32),
            jax.ShapeDtypeStruct((n, 16), jnp.float32),
        ],
    )(x, W1, Acat)


_mesh = plsc.VectorSubcoreMesh(
    core_axis_name="c", subcore_axis_name="s", num_cores=2, num_subcores=16)


@functools.partial(
    pl.kernel,
    out_type=[
        jax.ShapeDtypeStruct((EP * 8,), jnp.float32),
        jax.ShapeDtypeStruct((2, NP, 16), jnp.float32),
    ],
    mesh=_mesh,
    compiler_params=pltpu.CompilerParams(
        use_tc_tiling_on_sc=False, needs_layout_passes=False),
    scratch_types=[
        pltpu.VMEM((K,), jnp.int32),
        pltpu.VMEM((K,), jnp.int32),
        pltpu.VMEM((K, 16), jnp.float32),
        pltpu.VMEM((K, 16), jnp.float32),
        pltpu.VMEM((K, 16), jnp.float32),
        pltpu.VMEM((K * 8,), jnp.float32),
        pltpu.VMEM((64, 16), jnp.float32),
        pltpu.VMEM((16,), jnp.float32),
        pltpu.VMEM_SHARED((NP, 16), jnp.float32),
        pltpu.SemaphoreType.DMA,
    ],
)
def _edge1(asd_hbm, dsa_hbm, src_hbm, dst_hbm, shift_hbm,
           e_hbm, segp_hbm,
           srcv, dstv, srows, drows, ebuf, ecomp, zbuf, shiftbuf,
           seg_acc, sem):
    cid = lax.axis_index("c")
    sid = lax.axis_index("s")
    tile = cid * 16 + sid
    ebase = tile * EPT

    pltpu.sync_copy(shift_hbm, shiftbuf)

    zv = jnp.zeros((16,), jnp.float32)

    @pl.loop(0, 64)
    def _(r):
        zbuf[r, :] = zv

    row0 = sid * RPS

    @pl.loop(0, RPS // 64)
    def _(j):
        pltpu.sync_copy(zbuf, seg_acc.at[pl.ds(row0 + j * 64, 64)])

    @pl.loop(0, K)
    def _(r):
        ebuf[r, :] = zv

    plsc.subcore_barrier()

    shiftv = shiftbuf[...]
    lanes = lax.broadcasted_iota(jnp.int32, (16,), 0)
    headv = jnp.bitwise_and(lanes, 7)
    pat01 = lax.shift_right_logical(lanes, 3)

    @pl.loop(0, NCHUNK)
    def _(c):
        pltpu.sync_copy(src_hbm.at[tile, c], srcv)
        pltpu.sync_copy(dst_hbm.at[tile, c], dstv)
        pltpu.async_copy(asd_hbm.at[srcv], srows, sem).wait()
        pltpu.async_copy(dsa_hbm.at[dstv], drows, sem).wait()
        for i in range(K // 2):
            rowv = pat01 + (2 * i)
            sv = plsc.load_gather(srows, [rowv, headv])
            dv = plsc.load_gather(drows, [rowv, headv])
            al = sv + dv
            al = jnp.maximum(al, 0.2 * al)
            ev = jnp.exp(al - shiftv)
            plsc.store_scatter(ebuf, [rowv, headv], ev)
            ecomp[pl.ds(i * 16, 16)] = ev
        pltpu.sync_copy(ebuf, seg_acc.at[dstv], add=True)
        pltpu.sync_copy(ecomp, e_hbm.at[pl.ds((ebase + c * K) * 8, K * 8)])

    plsc.subcore_barrier()
    pltpu.sync_copy(seg_acc.at[pl.ds(row0, RPS)],
                    segp_hbm.at[cid, pl.ds(row0, RPS)])


def kernel(x, edge_index, W1, att_src1, att_dst1, b1, W2, att_src2, att_dst2, b2):
    n = N
    loop = jnp.arange(n, dtype=jnp.int32)
    src = jnp.concatenate([edge_index[0].astype(jnp.int32), loop])
    dst = jnp.concatenate([edge_index[1].astype(jnp.int32), loop])
    src_p = jnp.concatenate([src, jnp.full((EP - NE,), n, jnp.int32)])
    dst_p = jnp.concatenate([dst, jnp.full((EP - NE,), n, jnp.int32)])
    src2d = src_p.reshape(32, NCHUNK, K)
    dst2d = dst_p.reshape(32, NCHUNK, K)

    A_src = jnp.zeros((HEADS * HID, HEADS), jnp.float32)
    A_src = A_src.at[jnp.arange(HEADS * HID), jnp.arange(HEADS * HID) // HID].set(
        att_src1.reshape(-1))
    A_dst = jnp.zeros((HEADS * HID, HEADS), jnp.float32)
    A_dst = A_dst.at[jnp.arange(HEADS * HID), jnp.arange(HEADS * HID) // HID].set(
        att_dst1.reshape(-1))
    Acat = jnp.concatenate([A_src, A_dst], axis=1)

    h, asd = _node1(x, W1, Acat)
    shift8 = jnp.max(asd[:, :HEADS], axis=0) + jnp.max(asd[:, HEADS:], axis=0)
    shift16 = jnp.concatenate([shift8, shift8])

    asd_p = jnp.pad(asd, ((0, NP - n), (0, 0)))
    dsa_p = jnp.concatenate([asd_p[:, HEADS:], asd_p[:, :HEADS]], axis=1)

    e_flat, segp = _edge1(asd_p, dsa_p, src2d, dst2d, shift16)
    seg_sum = (segp[0] + segp[1])[:n, :HEADS]
    ealpha = e_flat.reshape(EP, HEADS)[:NE]

    w = ealpha / (seg_sum[dst] + 1e-16 * jnp.exp(-shift8)[None, :])
    hh = h.reshape(n, HEADS, HID)
    msg = hh[src] * w[:, :, None]
    out = jax.ops.segment_sum(msg, dst, num_segments=n).reshape(n, HEADS * HID)
    out = out + b1

    z = jax.nn.elu(out)
    h2 = (z @ W2)
    a_src2 = (h2 * att_src2[0]).sum(-1)
    a_dst2 = (h2 * att_dst2[0]).sum(-1)
    shift2 = jnp.max(a_src2) + jnp.max(a_dst2)
    alpha2 = a_src2[src] + a_dst2[dst]
    alpha2 = jax.nn.leaky_relu(alpha2, negative_slope=0.2)
    e2 = jnp.exp(alpha2 - shift2)
    seg2 = jax.ops.segment_sum(e2, dst, num_segments=n)
    w2 = e2 / (seg2[dst] + 1e-16 * jnp.exp(-shift2))
    out2 = jax.ops.segment_sum(w2 * h2[src, 0], dst, num_segments=n)
    out2 = out2 + b2[0]
    return jax.nn.sigmoid(out2)


# speedup vs baseline: 4.1622x; 3.7500x over previous
import functools

import jax
import jax.numpy as jnp
from jax import lax
from jax.experimental import pallas as pl
from jax.experimental.pallas import tpu as pltpu
from jax.experimental.pallas import tpu_sc as plsc

N = 10000
E = 320000
IN_CH = 128
HID = 64
HEADS = 8
OUT_CH = 1

NP = 10240
NE = E + N
EPT = 10496
EP = 32 * EPT
K = 128
NCHUNK = EPT // K
RPS = NP // 16

ROWS = 512

_SC_PARAMS = pltpu.CompilerParams(
    use_tc_tiling_on_sc=False, needs_layout_passes=False)

_mesh = plsc.VectorSubcoreMesh(
    core_axis_name="c", subcore_axis_name="s", num_cores=2, num_subcores=16)


def _node1_body(x_ref, w_ref, acat_ref, h_ref, asd_ref):
    p = pl.program_id(1)
    hp = jnp.dot(x_ref[...], w_ref[...], preferred_element_type=jnp.float32)
    h_ref[0] = hp
    contrib = jnp.dot(hp, acat_ref[...], preferred_element_type=jnp.float32)

    @pl.when(p == 0)
    def _():
        asd_ref[...] = contrib

    @pl.when(p != 0)
    def _():
        asd_ref[...] += contrib


def _node1(x_pad, W1, Acat):
    return pl.pallas_call(
        _node1_body,
        grid=(NP // ROWS, 4),
        in_specs=[
            pl.BlockSpec((ROWS, IN_CH), lambda i, p: (i, 0)),
            pl.BlockSpec((IN_CH, 128), lambda i, p: (0, p)),
            pl.BlockSpec((128, 16), lambda i, p: (p, 0)),
        ],
        out_specs=[
            pl.BlockSpec((1, ROWS, 128), lambda i, p: (p, i, 0)),
            pl.BlockSpec((ROWS, 16), lambda i, p: (i, 0)),
        ],
        out_shape=[
            jax.ShapeDtypeStruct((4, NP, 128), jnp.float32),
            jax.ShapeDtypeStruct((NP, 16), jnp.float32),
        ],
    )(x_pad, W1, Acat)


def _inv_body(segp_ref, eps_ref, inv_ref):
    inv_ref[...] = 1.0 / (segp_ref[0] + segp_ref[1] + eps_ref[...])


def _inv(segp, eps128):
    return pl.pallas_call(
        _inv_body,
        grid=(2,),
        in_specs=[
            pl.BlockSpec((2, NP * 16 // 256, 128), lambda i: (0, i, 0)),
            pl.BlockSpec((1, 128), lambda i: (0, 0)),
        ],
        out_specs=pl.BlockSpec((NP * 16 // 256, 128), lambda i: (i, 0)),
        out_shape=jax.ShapeDtypeStruct((NP * 16 // 128, 128), jnp.float32),
    )(segp.reshape(2, NP * 16 // 128, 128), eps128)


@functools.partial(
    pl.kernel,
    out_type=[
        jax.ShapeDtypeStruct((EP * 8,), jnp.float32),
        jax.ShapeDtypeStruct((2, NP, 16), jnp.float32),
    ],
    mesh=_mesh,
    compiler_params=_SC_PARAMS,
    scratch_types=[
        pltpu.VMEM((K,), jnp.int32),
        pltpu.VMEM((K,), jnp.int32),
        pltpu.VMEM((K, 16), jnp.float32),
        pltpu.VMEM((K, 16), jnp.float32),
        pltpu.VMEM((K, 16), jnp.float32),
        pltpu.VMEM((K * 8,), jnp.float32),
        pltpu.VMEM((64, 16), jnp.float32),
        pltpu.VMEM((16,), jnp.float32),
        pltpu.VMEM_SHARED((NP, 16), jnp.float32),
        pltpu.SemaphoreType.DMA,
    ],
)
def _edge1(asd_hbm, dsa_hbm, src_hbm, dst_hbm, shift_hbm,
           e_hbm, segp_hbm,
           srcv, dstv, srows, drows, ebuf, ecomp, zbuf, shiftbuf,
           seg_acc, sem):
    cid = lax.axis_index("c")
    sid = lax.axis_index("s")
    tile = cid * 16 + sid
    ebase = tile * EPT

    pltpu.sync_copy(shift_hbm, shiftbuf)

    zv = jnp.zeros((16,), jnp.float32)

    @pl.loop(0, 64)
    def _(r):
        zbuf[r, :] = zv

    row0 = sid * RPS

    @pl.loop(0, RPS // 64)
    def _(j):
        pltpu.sync_copy(zbuf, seg_acc.at[pl.ds(row0 + j * 64, 64)])

    @pl.loop(0, K)
    def _(r):
        ebuf[r, :] = zv

    plsc.subcore_barrier()

    shiftv = shiftbuf[...]
    lanes = lax.broadcasted_iota(jnp.int32, (16,), 0)
    headv = jnp.bitwise_and(lanes, 7)
    pat01 = lax.shift_right_logical(lanes, 3)

    @pl.loop(0, NCHUNK)
    def _(c):
        pltpu.sync_copy(src_hbm.at[tile, c], srcv)
        pltpu.sync_copy(dst_hbm.at[tile, c], dstv)
        pltpu.async_copy(asd_hbm.at[srcv], srows, sem).wait()
        pltpu.async_copy(dsa_hbm.at[dstv], drows, sem).wait()
        for i in range(K // 2):
            rowv = pat01 + (2 * i)
            sv = plsc.load_gather(srows, [rowv, headv])
            dv = plsc.load_gather(drows, [rowv, headv])
            al = sv + dv
            al = jnp.maximum(al, 0.2 * al)
            ev = jnp.exp(al - shiftv)
            plsc.store_scatter(ebuf, [rowv, headv], ev)
            ecomp[pl.ds(i * 16, 16)] = ev
        pltpu.sync_copy(ebuf, seg_acc.at[dstv], add=True)
        pltpu.sync_copy(ecomp, e_hbm.at[pl.ds((ebase + c * K) * 8, K * 8)])

    plsc.subcore_barrier()
    pltpu.sync_copy(seg_acc.at[pl.ds(row0, RPS)],
                    segp_hbm.at[cid, pl.ds(row0, RPS)])


@functools.partial(
    pl.kernel,
    out_type=[jax.ShapeDtypeStruct((4, 2, NP, 128), jnp.float32)],
    mesh=_mesh,
    compiler_params=_SC_PARAMS,
    scratch_types=[
        pltpu.VMEM((K,), jnp.int32),
        pltpu.VMEM((K,), jnp.int32),
        pltpu.VMEM((K, 128), jnp.float32),
        pltpu.VMEM((K, 128), jnp.float32),
        pltpu.VMEM((K, 16), jnp.float32),
        pltpu.VMEM((K * 8 + 16,), jnp.float32),
        pltpu.VMEM((16,), jnp.float32),
        pltpu.VMEM((64, 128), jnp.float32),
        pltpu.VMEM_SHARED((NP, 128), jnp.float32),
        pltpu.SemaphoreType.DMA,
        pltpu.SemaphoreType.DMA,
    ],
)
def _edge2(h4_hbm, inv_hbm, e_hbm, src_hbm, dst_hbm,
           out_hbm,
           srcv, dstv, hrows, msg, invrows, erow, wbuf, zbuf, acc,
           sem, sem2):
    cid = lax.axis_index("c")
    sid = lax.axis_index("s")
    tile = cid * 16 + sid
    ebase = tile * EPT
    row0 = sid * RPS

    lanes = lax.broadcasted_iota(jnp.int32, (16,), 0)
    pat01 = lax.shift_right_logical(lanes, 3)
    zero16 = jnp.zeros((16,), jnp.int32)
    eight16 = zero16 + 8
    zv = jnp.zeros((16,), jnp.float32)

    @pl.loop(0, 64)
    def _(r):
        for j in range(8):
            zbuf[r, pl.ds(j * 16, 16)] = zv

    for p in range(4):
        @pl.loop(0, RPS // 64)
        def _(j):
            pltpu.sync_copy(zbuf, acc.at[pl.ds(row0 + j * 64, 64)])

        plsc.subcore_barrier()

        @pl.loop(0, NCHUNK)
        def _(c):
            pltpu.sync_copy(src_hbm.at[tile, c], srcv)
            pltpu.sync_copy(dst_hbm.at[tile, c], dstv)
            pltpu.async_copy(h4_hbm.at[p].at[srcv], hrows, sem).wait()
            pltpu.async_copy(inv_hbm.at[dstv], invrows, sem2).wait()
            pltpu.sync_copy(e_hbm.at[pl.ds((ebase + c * K) * 8, K * 8)],
                            erow.at[pl.ds(0, K * 8)])

            @pl.loop(0, K)
            def _(e):
                ewv = erow[pl.ds(e * 8, 16)]
                iwv = invrows[e, :]
                w0 = ewv[2 * p] * iwv[2 * p]
                w1 = ewv[2 * p + 1] * iwv[2 * p + 1]
                for j in range(8):
                    hv = hrows[e, pl.ds(j * 16, 16)]
                    msg[e, pl.ds(j * 16, 16)] = hv * (w0 if j < 4 else w1)

            pltpu.sync_copy(msg, acc.at[dstv], add=True)

        plsc.subcore_barrier()
        pltpu.sync_copy(acc.at[pl.ds(row0, RPS)],
                        out_hbm.at[p, cid, pl.ds(row0, RPS)])


def kernel(x, edge_index, W1, att_src1, att_dst1, b1, W2, att_src2, att_dst2, b2):
    n = N
    loop = jnp.arange(n, dtype=jnp.int32)
    src = jnp.concatenate([edge_index[0].astype(jnp.int32), loop])
    dst = jnp.concatenate([edge_index[1].astype(jnp.int32), loop])
    src_p = jnp.concatenate([src, jnp.full((EP - NE,), n, jnp.int32)])
    dst_p = jnp.concatenate([dst, jnp.full((EP - NE,), n, jnp.int32)])
    src2d = src_p.reshape(32, NCHUNK, K)
    dst2d = dst_p.reshape(32, NCHUNK, K)

    A_src = jnp.zeros((HEADS * HID, HEADS), jnp.float32)
    A_src = A_src.at[jnp.arange(HEADS * HID), jnp.arange(HEADS * HID) // HID].set(
        att_src1.reshape(-1))
    A_dst = jnp.zeros((HEADS * HID, HEADS), jnp.float32)
    A_dst = A_dst.at[jnp.arange(HEADS * HID), jnp.arange(HEADS * HID) // HID].set(
        att_dst1.reshape(-1))
    Acat = jnp.concatenate([A_src, A_dst], axis=1)

    x_pad = jnp.pad(x, ((0, NP - n), (0, 0)))
    h4, asd = _node1(x_pad, W1, Acat)
    shift8 = jnp.max(asd[:n, :HEADS], axis=0) + jnp.max(asd[:n, HEADS:], axis=0)
    shift16 = jnp.concatenate([shift8, shift8])

    dsa = jnp.concatenate([asd[:, HEADS:], asd[:, :HEADS]], axis=1)

    e_flat, segp = _edge1(asd, dsa, src2d, dst2d, shift16)

    eps8 = 1e-16 * jnp.exp(-shift8)
    eps128 = jnp.tile(jnp.concatenate([eps8, eps8]), 8)[None, :]
    inv = _inv(segp, eps128).reshape(NP, 16)

    out4, = _edge2(h4, inv, e_flat, src2d, dst2d)
    out1 = (out4[:, 0] + out4[:, 1])
    out1 = jnp.moveaxis(out1, 0, 1).reshape(NP, 4 * 128)[:n]
    out = out1 + b1

    z = jax.nn.elu(out)
    h2 = (z @ W2)
    a_src2 = (h2 * att_src2[0]).sum(-1)
    a_dst2 = (h2 * att_dst2[0]).sum(-1)
    shift2 = jnp.max(a_src2) + jnp.max(a_dst2)
    alpha2 = a_src2[src] + a_dst2[dst]
    alpha2 = jax.nn.leaky_relu(alpha2, negative_slope=0.2)
    e2 = jnp.exp(alpha2 - shift2)
    seg2 = jax.ops.segment_sum(e2, dst, num_segments=n)
    w2 = e2 / (seg2[dst] + 1e-16 * jnp.exp(-shift2))
    out2 = jax.ops.segment_sum(w2 * h2[src, 0], dst, num_segments=n)
    out2 = out2 + b2[0]
    return jax.nn.sigmoid(out2)


# speedup vs baseline: 12.9806x; 3.1187x over previous
import functools

import jax
import jax.numpy as jnp
from jax import lax
from jax.experimental import pallas as pl
from jax.experimental.pallas import tpu as pltpu
from jax.experimental.pallas import tpu_sc as plsc

N = 10000
E = 320000
IN_CH = 128
HID = 64
HEADS = 8
OUT_CH = 1

NP = 10240
NE = E + N
EPT = 10496
EP = 32 * EPT
K = 128
NCHUNK = EPT // K
RPS = NP // 16

ROWS = 512

_SC_PARAMS = pltpu.CompilerParams(
    use_tc_tiling_on_sc=False, needs_layout_passes=False)

_mesh = plsc.VectorSubcoreMesh(
    core_axis_name="c", subcore_axis_name="s", num_cores=2, num_subcores=16)


def _node1_body(x_ref, w_ref, acat_ref, h_ref, asd_ref):
    p = pl.program_id(1)
    hp = jnp.dot(x_ref[...], w_ref[...], preferred_element_type=jnp.float32)
    h_ref[0] = hp
    contrib = jnp.dot(hp, acat_ref[...], preferred_element_type=jnp.float32)

    @pl.when(p == 0)
    def _():
        asd_ref[...] = contrib

    @pl.when(p != 0)
    def _():
        asd_ref[...] += contrib


def _node1(x_pad, W1, Acat):
    return pl.pallas_call(
        _node1_body,
        grid=(NP // ROWS, 4),
        in_specs=[
            pl.BlockSpec((ROWS, IN_CH), lambda i, p: (i, 0)),
            pl.BlockSpec((IN_CH, 128), lambda i, p: (0, p)),
            pl.BlockSpec((128, 16), lambda i, p: (p, 0)),
        ],
        out_specs=[
            pl.BlockSpec((1, ROWS, 128), lambda i, p: (p, i, 0)),
            pl.BlockSpec((ROWS, 16), lambda i, p: (i, 0)),
        ],
        out_shape=[
            jax.ShapeDtypeStruct((4, NP, 128), jnp.float32),
            jax.ShapeDtypeStruct((NP, 16), jnp.float32),
        ],
    )(x_pad, W1, Acat)


def _inv_body(segp_ref, eps_ref, inv_ref):
    inv_ref[...] = 1.0 / (segp_ref[0] + segp_ref[1] + eps_ref[...])


def _inv(segp, eps128):
    return pl.pallas_call(
        _inv_body,
        grid=(2,),
        in_specs=[
            pl.BlockSpec((2, NP * 16 // 256, 128), lambda i: (0, i, 0)),
            pl.BlockSpec((1, 128), lambda i: (0, 0)),
        ],
        out_specs=pl.BlockSpec((NP * 16 // 256, 128), lambda i: (i, 0)),
        out_shape=jax.ShapeDtypeStruct((NP * 16 // 128, 128), jnp.float32),
    )(segp.reshape(2, NP * 16 // 128, 128), eps128)


@functools.partial(
    pl.kernel,
    out_type=[
        jax.ShapeDtypeStruct((EP * 8,), jnp.float32),
        jax.ShapeDtypeStruct((2, NP, 16), jnp.float32),
    ],
    mesh=_mesh,
    compiler_params=_SC_PARAMS,
    scratch_types=[
        pltpu.VMEM((K,), jnp.int32),
        pltpu.VMEM((K,), jnp.int32),
        pltpu.VMEM((K, 16), jnp.float32),
        pltpu.VMEM((K, 16), jnp.float32),
        pltpu.VMEM((K, 16), jnp.float32),
        pltpu.VMEM((K * 8,), jnp.float32),
        pltpu.VMEM((64, 16), jnp.float32),
        pltpu.VMEM((16,), jnp.float32),
        pltpu.VMEM_SHARED((NP, 16), jnp.float32),
        pltpu.SemaphoreType.DMA,
    ],
)
def _edge1(asd_hbm, dsa_hbm, src_hbm, dst_hbm, shift_hbm,
           e_hbm, segp_hbm,
           srcv, dstv, srows, drows, ebuf, ecomp, zbuf, shiftbuf,
           seg_acc, sem):
    cid = lax.axis_index("c")
    sid = lax.axis_index("s")
    tile = cid * 16 + sid
    ebase = tile * EPT

    pltpu.sync_copy(shift_hbm, shiftbuf)

    zv = jnp.zeros((16,), jnp.float32)

    @pl.loop(0, 64)
    def _(r):
        zbuf[r, :] = zv

    row0 = sid * RPS

    @pl.loop(0, RPS // 64)
    def _(j):
        pltpu.sync_copy(zbuf, seg_acc.at[pl.ds(row0 + j * 64, 64)])

    @pl.loop(0, K)
    def _(r):
        ebuf[r, :] = zv

    plsc.subcore_barrier()

    shiftv = shiftbuf[...]
    lanes = lax.broadcasted_iota(jnp.int32, (16,), 0)
    headv = jnp.bitwise_and(lanes, 7)
    pat01 = lax.shift_right_logical(lanes, 3)

    @pl.loop(0, NCHUNK)
    def _(c):
        pltpu.sync_copy(src_hbm.at[tile, c], srcv)
        pltpu.sync_copy(dst_hbm.at[tile, c], dstv)
        pltpu.async_copy(asd_hbm.at[srcv], srows, sem).wait()
        pltpu.async_copy(dsa_hbm.at[dstv], drows, sem).wait()
        for i in range(K // 2):
            rowv = pat01 + (2 * i)
            sv = plsc.load_gather(srows, [rowv, headv])
            dv = plsc.load_gather(drows, [rowv, headv])
            al = sv + dv
            al = jnp.maximum(al, 0.2 * al)
            ev = jnp.exp(al - shiftv)
            plsc.store_scatter(ebuf, [rowv, headv], ev)
            ecomp[pl.ds(i * 16, 16)] = ev
        pltpu.sync_copy(ebuf, seg_acc.at[dstv], add=True)
        pltpu.sync_copy(ecomp, e_hbm.at[pl.ds((ebase + c * K) * 8, K * 8)])

    plsc.subcore_barrier()
    pltpu.sync_copy(seg_acc.at[pl.ds(row0, RPS)],
                    segp_hbm.at[cid, pl.ds(row0, RPS)])


def _node2_body(out4_ref, b1_ref, w2e_ref, h2a_ref):
    p = pl.program_id(1)
    blk = out4_ref[0]
    z = blk[0] + blk[1] + b1_ref[0]
    z = jnp.where(z > 0, z, jnp.exp(jnp.minimum(z, 0.0)) - 1.0)
    contrib = jnp.dot(z, w2e_ref[...], preferred_element_type=jnp.float32)

    @pl.when(p == 0)
    def _():
        h2a_ref[...] = contrib

    @pl.when(p != 0)
    def _():
        h2a_ref[...] += contrib


def _node2(out4, b1r, W2ext):
    return pl.pallas_call(
        _node2_body,
        grid=(NP // ROWS, 4),
        in_specs=[
            pl.BlockSpec((1, 2, ROWS, 128), lambda i, p: (p, 0, i, 0)),
            pl.BlockSpec((1, 1, 128), lambda i, p: (p, 0, 0)),
            pl.BlockSpec((128, 128), lambda i, p: (p, 0)),
        ],
        out_specs=pl.BlockSpec((ROWS, 128), lambda i, p: (i, 0)),
        out_shape=jax.ShapeDtypeStruct((NP, 128), jnp.float32),
    )(out4, b1r, W2ext)


def _fin_body(p_ref, b2_ref, o_ref):
    o_ref[...] = jax.nn.sigmoid(p_ref[0] + p_ref[1] + b2_ref[...])


def _fin(out2p, b2arr):
    return pl.pallas_call(
        _fin_body,
        grid=(2,),
        in_specs=[
            pl.BlockSpec((2, NP * 16 // 256, 128), lambda i: (0, i, 0)),
            pl.BlockSpec((1, 128), lambda i: (0, 0)),
        ],
        out_specs=pl.BlockSpec((NP * 16 // 256, 128), lambda i: (i, 0)),
        out_shape=jax.ShapeDtypeStruct((NP * 16 // 128, 128), jnp.float32),
    )(out2p.reshape(2, NP * 16 // 128, 128), b2arr)


@functools.partial(
    pl.kernel,
    out_type=[
        jax.ShapeDtypeStruct((EP,), jnp.float32),
        jax.ShapeDtypeStruct((2, NP, 16), jnp.float32),
    ],
    mesh=_mesh,
    compiler_params=_SC_PARAMS,
    scratch_types=[
        pltpu.VMEM((NP,), jnp.float32),
        pltpu.VMEM((NP,), jnp.float32),
        pltpu.VMEM((K,), jnp.int32),
        pltpu.VMEM((K,), jnp.int32),
        pltpu.VMEM((K, 16), jnp.float32),
        pltpu.VMEM((K,), jnp.float32),
        pltpu.VMEM((64, 16), jnp.float32),
        pltpu.VMEM((16,), jnp.float32),
        pltpu.VMEM_SHARED((NP, 16), jnp.float32),
    ],
)
def _edge3a(asrc2_hbm, adst2_hbm, src_hbm, dst_hbm, shift2_hbm,
            e2_hbm, seg2p_hbm,
            at, dt, srcv, dstv, ebuf, e2c, zbuf, shiftbuf, acc2):
    cid = lax.axis_index("c")
    sid = lax.axis_index("s")
    tile = cid * 16 + sid
    ebase = tile * EPT
    row0 = sid * RPS

    pltpu.sync_copy(asrc2_hbm, at)
    pltpu.sync_copy(adst2_hbm, dt)
    pltpu.sync_copy(shift2_hbm, shiftbuf)

    zv = jnp.zeros((16,), jnp.float32)

    @pl.loop(0, 64)
    def _(r):
        zbuf[r, :] = zv

    @pl.loop(0, RPS // 64)
    def _(j):
        pltpu.sync_copy(zbuf, acc2.at[pl.ds(row0 + j * 64, 64)])

    @pl.loop(0, K)
    def _(r):
        ebuf[r, :] = zv

    plsc.subcore_barrier()

    s2v = shiftbuf[...]
    lanes = lax.broadcasted_iota(jnp.int32, (16,), 0)
    zero16 = jnp.zeros((16,), jnp.int32)

    @pl.loop(0, NCHUNK)
    def _(c):
        pltpu.sync_copy(src_hbm.at[tile, c], srcv)
        pltpu.sync_copy(dst_hbm.at[tile, c], dstv)
        for i in range(K // 16):
            src16 = srcv[pl.ds(i * 16, 16)]
            dst16 = dstv[pl.ds(i * 16, 16)]
            sv = plsc.load_gather(at, [src16])
            dv = plsc.load_gather(dt, [dst16])
            al = sv + dv
            al = jnp.maximum(al, 0.2 * al)
            e2v = jnp.exp(al - s2v)
            e2c[pl.ds(i * 16, 16)] = e2v
            plsc.store_scatter(ebuf, [lanes + i * 16, zero16], e2v)
        pltpu.sync_copy(ebuf, acc2.at[dstv], add=True)
        pltpu.sync_copy(e2c, e2_hbm.at[pl.ds(ebase + c * K, K)])

    plsc.subcore_barrier()
    pltpu.sync_copy(acc2.at[pl.ds(row0, RPS)],
                    seg2p_hbm.at[cid, pl.ds(row0, RPS)])


@functools.partial(
    pl.kernel,
    out_type=[jax.ShapeDtypeStruct((2, NP, 16), jnp.float32)],
    mesh=_mesh,
    compiler_params=_SC_PARAMS,
    scratch_types=[
        pltpu.VMEM((NP,), jnp.float32),
        pltpu.VMEM((NP,), jnp.float32),
        pltpu.VMEM((K,), jnp.int32),
        pltpu.VMEM((K,), jnp.int32),
        pltpu.VMEM((K,), jnp.float32),
        pltpu.VMEM((K, 16), jnp.float32),
        pltpu.VMEM((64, 16), jnp.float32),
        pltpu.VMEM_SHARED((NP, 16), jnp.float32),
    ],
)
def _edge3b(h2_hbm, inv2_hbm, e2_hbm, src_hbm, dst_hbm,
            out2p_hbm,
            h2t, invt, srcv, dstv, e2c, mrow, zbuf, acc2):
    cid = lax.axis_index("c")
    sid = lax.axis_index("s")
    tile = cid * 16 + sid
    ebase = tile * EPT
    row0 = sid * RPS

    pltpu.sync_copy(h2_hbm, h2t)
    pltpu.sync_copy(inv2_hbm, invt)

    zv = jnp.zeros((16,), jnp.float32)

    @pl.loop(0, 64)
    def _(r):
        zbuf[r, :] = zv

    @pl.loop(0, RPS // 64)
    def _(j):
        pltpu.sync_copy(zbuf, acc2.at[pl.ds(row0 + j * 64, 64)])

    @pl.loop(0, K)
    def _(r):
        mrow[r, :] = zv

    plsc.subcore_barrier()

    lanes = lax.broadcasted_iota(jnp.int32, (16,), 0)
    zero16 = jnp.zeros((16,), jnp.int32)

    @pl.loop(0, NCHUNK)
    def _(c):
        pltpu.sync_copy(src_hbm.at[tile, c], srcv)
        pltpu.sync_copy(dst_hbm.at[tile, c], dstv)
        pltpu.sync_copy(e2_hbm.at[pl.ds(ebase + c * K, K)], e2c)
        for i in range(K // 16):
            src16 = srcv[pl.ds(i * 16, 16)]
            dst16 = dstv[pl.ds(i * 16, 16)]
            e2v = e2c[pl.ds(i * 16, 16)]
            w = e2v * plsc.load_gather(invt, [dst16])
            m = w * plsc.load_gather(h2t, [src16])
            plsc.store_scatter(mrow, [lanes + i * 16, zero16], m)
        pltpu.sync_copy(mrow, acc2.at[dstv], add=True)

    plsc.subcore_barrier()
    pltpu.sync_copy(acc2.at[pl.ds(row0, RPS)],
                    out2p_hbm.at[cid, pl.ds(row0, RPS)])


@functools.partial(
    pl.kernel,
    out_type=[jax.ShapeDtypeStruct((4, 2, NP, 128), jnp.float32)],
    mesh=_mesh,
    compiler_params=_SC_PARAMS,
    scratch_types=[
        pltpu.VMEM((K,), jnp.int32),
        pltpu.VMEM((K,), jnp.int32),
        pltpu.VMEM((K, 128), jnp.float32),
        pltpu.VMEM((K, 128), jnp.float32),
        pltpu.VMEM((K, 16), jnp.float32),
        pltpu.VMEM((K * 8 + 16,), jnp.float32),
        pltpu.VMEM((16,), jnp.float32),
        pltpu.VMEM((64, 128), jnp.float32),
        pltpu.VMEM_SHARED((NP, 128), jnp.float32),
        pltpu.SemaphoreType.DMA,
        pltpu.SemaphoreType.DMA,
    ],
)
def _edge2(h4_hbm, inv_hbm, e_hbm, src_hbm, dst_hbm,
           out_hbm,
           srcv, dstv, hrows, msg, invrows, erow, wbuf, zbuf, acc,
           sem, sem2):
    cid = lax.axis_index("c")
    sid = lax.axis_index("s")
    tile = cid * 16 + sid
    ebase = tile * EPT
    row0 = sid * RPS

    lanes = lax.broadcasted_iota(jnp.int32, (16,), 0)
    pat01 = lax.shift_right_logical(lanes, 3)
    zero16 = jnp.zeros((16,), jnp.int32)
    eight16 = zero16 + 8
    zv = jnp.zeros((16,), jnp.float32)

    @pl.loop(0, 64)
    def _(r):
        for j in range(8):
            zbuf[r, pl.ds(j * 16, 16)] = zv

    for p in range(4):
        @pl.loop(0, RPS // 64)
        def _(j):
            pltpu.sync_copy(zbuf, acc.at[pl.ds(row0 + j * 64, 64)])

        plsc.subcore_barrier()

        @pl.loop(0, NCHUNK)
        def _(c):
            pltpu.sync_copy(src_hbm.at[tile, c], srcv)
            pltpu.sync_copy(dst_hbm.at[tile, c], dstv)
            pltpu.async_copy(h4_hbm.at[p].at[srcv], hrows, sem).wait()
            pltpu.async_copy(inv_hbm.at[dstv], invrows, sem2).wait()
            pltpu.sync_copy(e_hbm.at[pl.ds((ebase + c * K) * 8, K * 8)],
                            erow.at[pl.ds(0, K * 8)])

            @pl.loop(0, K)
            def _(e):
                ewv = erow[pl.ds(e * 8, 16)]
                iwv = invrows[e, :]
                w0 = ewv[2 * p] * iwv[2 * p]
                w1 = ewv[2 * p + 1] * iwv[2 * p + 1]
                for j in range(8):
                    hv = hrows[e, pl.ds(j * 16, 16)]
                    msg[e, pl.ds(j * 16, 16)] = hv * (w0 if j < 4 else w1)

            pltpu.sync_copy(msg, acc.at[dstv], add=True)

        plsc.subcore_barrier()
        pltpu.sync_copy(acc.at[pl.ds(row0, RPS)],
                        out_hbm.at[p, cid, pl.ds(row0, RPS)])


def kernel(x, edge_index, W1, att_src1, att_dst1, b1, W2, att_src2, att_dst2, b2):
    n = N
    loop = jnp.arange(n, dtype=jnp.int32)
    src = jnp.concatenate([edge_index[0].astype(jnp.int32), loop])
    dst = jnp.concatenate([edge_index[1].astype(jnp.int32), loop])
    src_p = jnp.concatenate([src, jnp.full((EP - NE,), n, jnp.int32)])
    dst_p = jnp.concatenate([dst, jnp.full((EP - NE,), n, jnp.int32)])
    src2d = src_p.reshape(32, NCHUNK, K)
    dst2d = dst_p.reshape(32, NCHUNK, K)

    A_src = jnp.zeros((HEADS * HID, HEADS), jnp.float32)
    A_src = A_src.at[jnp.arange(HEADS * HID), jnp.arange(HEADS * HID) // HID].set(
        att_src1.reshape(-1))
    A_dst = jnp.zeros((HEADS * HID, HEADS), jnp.float32)
    A_dst = A_dst.at[jnp.arange(HEADS * HID), jnp.arange(HEADS * HID) // HID].set(
        att_dst1.reshape(-1))
    Acat = jnp.concatenate([A_src, A_dst], axis=1)

    x_pad = jnp.pad(x, ((0, NP - n), (0, 0)))
    h4, asd = _node1(x_pad, W1, Acat)
    shift8 = jnp.max(asd[:n, :HEADS], axis=0) + jnp.max(asd[:n, HEADS:], axis=0)
    shift16 = jnp.concatenate([shift8, shift8])

    dsa = jnp.concatenate([asd[:, HEADS:], asd[:, :HEADS]], axis=1)

    e_flat, segp = _edge1(asd, dsa, src2d, dst2d, shift16)

    eps8 = 1e-16 * jnp.exp(-shift8)
    eps128 = jnp.tile(jnp.concatenate([eps8, eps8]), 8)[None, :]
    inv = _inv(segp, eps128).reshape(NP, 16)

    out4, = _edge2(h4, inv, e_flat, src2d, dst2d)

    W2ext = jnp.zeros((HEADS * HID, 128), jnp.float32)
    W2ext = W2ext.at[:, 0].set(W2[:, 0])
    W2ext = W2ext.at[:, 1].set(W2[:, 0] * att_src2[0, 0])
    W2ext = W2ext.at[:, 2].set(W2[:, 0] * att_dst2[0, 0])
    b1r = b1.reshape(4, 1, 128)
    h2a = _node2(out4, b1r, W2ext)
    h2v = h2a[:, 0]
    a_src2 = h2a[:, 1]
    a_dst2 = h2a[:, 2]
    shift2 = jnp.max(a_src2[:n]) + jnp.max(a_dst2[:n])
    shift2v = jnp.full((16,), shift2, jnp.float32)

    e2_flat, seg2p = _edge3a(a_src2, a_dst2, src2d, dst2d, shift2v)
    eps2 = 1e-16 * jnp.exp(-shift2)
    eps2_128 = jnp.full((1, 128), eps2, jnp.float32)
    inv2 = _inv(seg2p, eps2_128).reshape(NP, 16)[:, 0]
    out2p, = _edge3b(h2v, inv2, e2_flat, src2d, dst2d)

    b2arr = jnp.full((1, 128), b2[0], jnp.float32)
    sig = _fin(out2p, b2arr)
    return sig.reshape(NP, 16)[:n, 0]


# speedup vs baseline: 21.7116x; 1.6726x over previous
import functools

import jax
import jax.numpy as jnp
from jax import lax
from jax.experimental import pallas as pl
from jax.experimental.pallas import tpu as pltpu
from jax.experimental.pallas import tpu_sc as plsc

N = 10000
E = 320000
IN_CH = 128
HID = 64
HEADS = 8
OUT_CH = 1

NP = 10240
NE = E + N
EPT = 10496
EP = 32 * EPT
K = 128
NCHUNK = EPT // K
K2 = 64
NCHUNK2 = EPT // K2
RPS = NP // 16

ROWS = 512

_SC_PARAMS = pltpu.CompilerParams(
    use_tc_tiling_on_sc=False, needs_layout_passes=False)

_mesh = plsc.VectorSubcoreMesh(
    core_axis_name="c", subcore_axis_name="s", num_cores=2, num_subcores=16)


def _node1_body(x_ref, w_ref, acat_ref, h_ref, asd_ref):
    p = pl.program_id(1)
    hp = jnp.dot(x_ref[...], w_ref[...], preferred_element_type=jnp.float32)
    h_ref[0] = hp
    contrib = jnp.dot(hp, acat_ref[...], preferred_element_type=jnp.float32)

    @pl.when(p == 0)
    def _():
        asd_ref[...] = contrib

    @pl.when(p != 0)
    def _():
        asd_ref[...] += contrib


def _node1(x_pad, W1, Acat):
    return pl.pallas_call(
        _node1_body,
        grid=(NP // ROWS, 4),
        in_specs=[
            pl.BlockSpec((ROWS, IN_CH), lambda i, p: (i, 0)),
            pl.BlockSpec((IN_CH, 128), lambda i, p: (0, p)),
            pl.BlockSpec((128, 16), lambda i, p: (p, 0)),
        ],
        out_specs=[
            pl.BlockSpec((1, ROWS, 128), lambda i, p: (p, i, 0)),
            pl.BlockSpec((ROWS, 16), lambda i, p: (i, 0)),
        ],
        out_shape=[
            jax.ShapeDtypeStruct((4, NP, 128), jnp.float32),
            jax.ShapeDtypeStruct((NP, 16), jnp.float32),
        ],
    )(x_pad, W1, Acat)


def _inv_body(segp_ref, eps_ref, inv_ref):
    inv_ref[...] = 1.0 / (segp_ref[0] + segp_ref[1] + eps_ref[...])


def _inv(segp, eps128):
    return pl.pallas_call(
        _inv_body,
        grid=(2,),
        in_specs=[
            pl.BlockSpec((2, NP * 16 // 256, 128), lambda i: (0, i, 0)),
            pl.BlockSpec((1, 128), lambda i: (0, 0)),
        ],
        out_specs=pl.BlockSpec((NP * 16 // 256, 128), lambda i: (i, 0)),
        out_shape=jax.ShapeDtypeStruct((NP * 16 // 128, 128), jnp.float32),
    )(segp.reshape(2, NP * 16 // 128, 128), eps128)


@functools.partial(
    pl.kernel,
    out_type=[
        jax.ShapeDtypeStruct((EP * 8,), jnp.float32),
        jax.ShapeDtypeStruct((2, NP, 16), jnp.float32),
    ],
    mesh=_mesh,
    compiler_params=_SC_PARAMS,
    scratch_types=[
        pltpu.VMEM((K,), jnp.int32),
        pltpu.VMEM((K,), jnp.int32),
        pltpu.VMEM((K, 16), jnp.float32),
        pltpu.VMEM((K, 16), jnp.float32),
        pltpu.VMEM((K, 16), jnp.float32),
        pltpu.VMEM((K * 8,), jnp.float32),
        pltpu.VMEM((64, 16), jnp.float32),
        pltpu.VMEM((16,), jnp.float32),
        pltpu.VMEM_SHARED((NP, 16), jnp.float32),
        pltpu.SemaphoreType.DMA,
    ],
)
def _edge1(asd_hbm, dsa_hbm, src_hbm, dst_hbm, shift_hbm,
           e_hbm, segp_hbm,
           srcv, dstv, srows, drows, ebuf, ecomp, zbuf, shiftbuf,
           seg_acc, sem):
    cid = lax.axis_index("c")
    sid = lax.axis_index("s")
    tile = cid * 16 + sid
    ebase = tile * EPT

    pltpu.sync_copy(shift_hbm, shiftbuf)

    zv = jnp.zeros((16,), jnp.float32)

    @pl.loop(0, 64)
    def _(r):
        zbuf[r, :] = zv

    row0 = sid * RPS

    @pl.loop(0, RPS // 64)
    def _(j):
        pltpu.sync_copy(zbuf, seg_acc.at[pl.ds(row0 + j * 64, 64)])

    @pl.loop(0, K)
    def _(r):
        ebuf[r, :] = zv

    plsc.subcore_barrier()

    shiftv = shiftbuf[...]
    lanes = lax.broadcasted_iota(jnp.int32, (16,), 0)
    headv = jnp.bitwise_and(lanes, 7)
    pat01 = lax.shift_right_logical(lanes, 3)

    @pl.loop(0, NCHUNK)
    def _(c):
        pltpu.sync_copy(src_hbm.at[tile, c], srcv)
        pltpu.sync_copy(dst_hbm.at[tile, c], dstv)
        pltpu.async_copy(asd_hbm.at[srcv], srows, sem).wait()
        pltpu.async_copy(dsa_hbm.at[dstv], drows, sem).wait()
        for i in range(K // 2):
            rowv = pat01 + (2 * i)
            sv = plsc.load_gather(srows, [rowv, headv])
            dv = plsc.load_gather(drows, [rowv, headv])
            al = sv + dv
            al = jnp.maximum(al, 0.2 * al)
            ev = jnp.exp(al - shiftv)
            plsc.store_scatter(ebuf, [rowv, headv], ev)
            ecomp[pl.ds(i * 16, 16)] = ev
        pltpu.sync_copy(ebuf, seg_acc.at[dstv], add=True)
        pltpu.sync_copy(ecomp, e_hbm.at[pl.ds((ebase + c * K) * 8, K * 8)])

    plsc.subcore_barrier()
    pltpu.sync_copy(seg_acc.at[pl.ds(row0, RPS)],
                    segp_hbm.at[cid, pl.ds(row0, RPS)])


def _node2_body(out4_ref, invex_ref, b1_ref, w2e_ref, h2a_ref):
    p = pl.program_id(1)
    blk = out4_ref[0]
    z = invex_ref[0] * (blk[0] + blk[1]) + b1_ref[0]
    z = jnp.where(z > 0, z, jnp.exp(jnp.minimum(z, 0.0)) - 1.0)
    contrib = jnp.dot(z, w2e_ref[...], preferred_element_type=jnp.float32)

    @pl.when(p == 0)
    def _():
        h2a_ref[...] = contrib

    @pl.when(p != 0)
    def _():
        h2a_ref[...] += contrib


def _node2(out4, invex, b1r, W2ext):
    return pl.pallas_call(
        _node2_body,
        grid=(NP // ROWS, 4),
        in_specs=[
            pl.BlockSpec((1, 2, ROWS, 128), lambda i, p: (p, 0, i, 0)),
            pl.BlockSpec((1, ROWS, 128), lambda i, p: (p, i, 0)),
            pl.BlockSpec((1, 1, 128), lambda i, p: (p, 0, 0)),
            pl.BlockSpec((128, 128), lambda i, p: (p, 0)),
        ],
        out_specs=pl.BlockSpec((ROWS, 128), lambda i, p: (i, 0)),
        out_shape=jax.ShapeDtypeStruct((NP, 128), jnp.float32),
    )(out4, invex, b1r, W2ext)


def _invex_body(segp_ref, eps_ref, sel_ref, out_ref):
    s = segp_ref[0] + segp_ref[1]
    iv = 1.0 / (s + eps_ref[...])
    out_ref[0] = jnp.dot(iv, sel_ref[0], preferred_element_type=jnp.float32)


def _invex(segp, eps16, SelAll):
    return pl.pallas_call(
        _invex_body,
        grid=(4, NP // 512),
        in_specs=[
            pl.BlockSpec((2, 512, 16), lambda p, i: (0, i, 0)),
            pl.BlockSpec((1, 16), lambda p, i: (0, 0)),
            pl.BlockSpec((1, 16, 128), lambda p, i: (p, 0, 0)),
        ],
        out_specs=pl.BlockSpec((1, 512, 128), lambda p, i: (p, i, 0)),
        out_shape=jax.ShapeDtypeStruct((4, NP, 128), jnp.float32),
    )(segp, eps16, SelAll)


def _fin_body(p_ref, inv_ref, b2_ref, o_ref):
    o_ref[...] = jax.nn.sigmoid(
        inv_ref[...] * (p_ref[0] + p_ref[1]) + b2_ref[...])


def _fin(out2p, inv2full, b2arr):
    return pl.pallas_call(
        _fin_body,
        grid=(2,),
        in_specs=[
            pl.BlockSpec((2, NP * 16 // 256, 128), lambda i: (0, i, 0)),
            pl.BlockSpec((NP * 16 // 256, 128), lambda i: (i, 0)),
            pl.BlockSpec((1, 128), lambda i: (0, 0)),
        ],
        out_specs=pl.BlockSpec((NP * 16 // 256, 128), lambda i: (i, 0)),
        out_shape=jax.ShapeDtypeStruct((NP * 16 // 128, 128), jnp.float32),
    )(out2p.reshape(2, NP * 16 // 128, 128), inv2full, b2arr)


@functools.partial(
    pl.kernel,
    out_type=[
        jax.ShapeDtypeStruct((EP,), jnp.float32),
        jax.ShapeDtypeStruct((2, NP, 16), jnp.float32),
    ],
    mesh=_mesh,
    compiler_params=_SC_PARAMS,
    scratch_types=[
        pltpu.VMEM((NP,), jnp.float32),
        pltpu.VMEM((NP,), jnp.float32),
        pltpu.VMEM((K,), jnp.int32),
        pltpu.VMEM((K,), jnp.int32),
        pltpu.VMEM((K, 16), jnp.float32),
        pltpu.VMEM((K,), jnp.float32),
        pltpu.VMEM((64, 16), jnp.float32),
        pltpu.VMEM((16,), jnp.float32),
        pltpu.VMEM_SHARED((NP, 16), jnp.float32),
    ],
)
def _edge3a(asrc2_hbm, adst2_hbm, src_hbm, dst_hbm, shift2_hbm,
            e2_hbm, seg2p_hbm,
            at, dt, srcv, dstv, ebuf, e2c, zbuf, shiftbuf, acc2):
    cid = lax.axis_index("c")
    sid = lax.axis_index("s")
    tile = cid * 16 + sid
    ebase = tile * EPT
    row0 = sid * RPS

    pltpu.sync_copy(asrc2_hbm, at)
    pltpu.sync_copy(adst2_hbm, dt)
    pltpu.sync_copy(shift2_hbm, shiftbuf)

    zv = jnp.zeros((16,), jnp.float32)

    @pl.loop(0, 64)
    def _(r):
        zbuf[r, :] = zv

    @pl.loop(0, RPS // 64)
    def _(j):
        pltpu.sync_copy(zbuf, acc2.at[pl.ds(row0 + j * 64, 64)])

    @pl.loop(0, K)
    def _(r):
        ebuf[r, :] = zv

    plsc.subcore_barrier()

    s2v = shiftbuf[...]
    lanes = lax.broadcasted_iota(jnp.int32, (16,), 0)
    zero16 = jnp.zeros((16,), jnp.int32)

    @pl.loop(0, NCHUNK)
    def _(c):
        pltpu.sync_copy(src_hbm.at[tile, c], srcv)
        pltpu.sync_copy(dst_hbm.at[tile, c], dstv)
        for i in range(K // 16):
            src16 = srcv[pl.ds(i * 16, 16)]
            dst16 = dstv[pl.ds(i * 16, 16)]
            sv = plsc.load_gather(at, [src16])
            dv = plsc.load_gather(dt, [dst16])
            al = sv + dv
            al = jnp.maximum(al, 0.2 * al)
            e2v = jnp.exp(al - s2v)
            e2c[pl.ds(i * 16, 16)] = e2v
            plsc.store_scatter(ebuf, [lanes + i * 16, zero16], e2v)
        pltpu.sync_copy(ebuf, acc2.at[dstv], add=True)
        pltpu.sync_copy(e2c, e2_hbm.at[pl.ds(ebase + c * K, K)])

    plsc.subcore_barrier()
    pltpu.sync_copy(acc2.at[pl.ds(row0, RPS)],
                    seg2p_hbm.at[cid, pl.ds(row0, RPS)])


@functools.partial(
    pl.kernel,
    out_type=[jax.ShapeDtypeStruct((2, NP, 16), jnp.float32)],
    mesh=_mesh,
    compiler_params=_SC_PARAMS,
    scratch_types=[
        pltpu.VMEM((NP,), jnp.float32),
        pltpu.VMEM((K,), jnp.int32),
        pltpu.VMEM((K,), jnp.int32),
        pltpu.VMEM((K,), jnp.float32),
        pltpu.VMEM((K, 16), jnp.float32),
        pltpu.VMEM((64, 16), jnp.float32),
        pltpu.VMEM_SHARED((NP, 16), jnp.float32),
    ],
)
def _edge3b(h2_hbm, e2_hbm, src_hbm, dst_hbm,
            out2p_hbm,
            h2t, srcv, dstv, e2c, mrow, zbuf, acc2):
    cid = lax.axis_index("c")
    sid = lax.axis_index("s")
    tile = cid * 16 + sid
    ebase = tile * EPT
    row0 = sid * RPS

    pltpu.sync_copy(h2_hbm, h2t)

    zv = jnp.zeros((16,), jnp.float32)

    @pl.loop(0, 64)
    def _(r):
        zbuf[r, :] = zv

    @pl.loop(0, RPS // 64)
    def _(j):
        pltpu.sync_copy(zbuf, acc2.at[pl.ds(row0 + j * 64, 64)])

    @pl.loop(0, K)
    def _(r):
        mrow[r, :] = zv

    plsc.subcore_barrier()

    lanes = lax.broadcasted_iota(jnp.int32, (16,), 0)
    zero16 = jnp.zeros((16,), jnp.int32)

    @pl.loop(0, NCHUNK)
    def _(c):
        pltpu.sync_copy(src_hbm.at[tile, c], srcv)
        pltpu.sync_copy(dst_hbm.at[tile, c], dstv)
        pltpu.sync_copy(e2_hbm.at[pl.ds(ebase + c * K, K)], e2c)
        for i in range(K // 16):
            src16 = srcv[pl.ds(i * 16, 16)]
            dst16 = dstv[pl.ds(i * 16, 16)]
            e2v = e2c[pl.ds(i * 16, 16)]
            m = e2v * plsc.load_gather(h2t, [src16])
            plsc.store_scatter(mrow, [lanes + i * 16, zero16], m)
        pltpu.sync_copy(mrow, acc2.at[dstv], add=True)

    plsc.subcore_barrier()
    pltpu.sync_copy(acc2.at[pl.ds(row0, RPS)],
                    out2p_hbm.at[cid, pl.ds(row0, RPS)])


@functools.partial(
    pl.kernel,
    out_type=[jax.ShapeDtypeStruct((4, 2, NP, 128), jnp.float32)],
    mesh=_mesh,
    compiler_params=_SC_PARAMS,
    scratch_types=[
        pltpu.VMEM((NCHUNK2, K2), jnp.int32),
        pltpu.VMEM((NCHUNK2, K2), jnp.int32),
        pltpu.VMEM((K2, 128), jnp.float32),
        pltpu.VMEM((K2, 128), jnp.float32),
        pltpu.VMEM((K2 * 8 + 16,), jnp.float32),
        pltpu.VMEM((K2 * 8 + 16,), jnp.float32),
        pltpu.VMEM((K2, 128), jnp.float32),
        pltpu.VMEM_SHARED((NP, 128), jnp.float32),
        pltpu.SemaphoreType.DMA,
        pltpu.SemaphoreType.DMA,
        pltpu.SemaphoreType.DMA,
        pltpu.SemaphoreType.DMA,
    ],
)
def _edge2(h4_hbm, e_hbm, src_hbm, dst_hbm,
           out_hbm,
           srcall, dstall, hrows0, hrows1, erow0, erow1, msg, acc,
           semh0, semh1, seme0, seme1):
    cid = lax.axis_index("c")
    sid = lax.axis_index("s")
    tile = cid * 16 + sid
    ebase = tile * EPT
    row0 = sid * RPS

    pltpu.sync_copy(src_hbm.at[tile], srcall)
    pltpu.sync_copy(dst_hbm.at[tile], dstall)

    hb = [hrows0, hrows1]
    eb = [erow0, erow1]
    sh = [semh0, semh1]
    se = [seme0, seme1]
    zv = jnp.zeros((16,), jnp.float32)

    def _e_src(c):
        return e_hbm.at[pl.ds((ebase + c * K2) * 8, K2 * 8)]

    for p in range(4):
        @pl.loop(0, K2)
        def _(r):
            for j in range(8):
                msg[r, pl.ds(j * 16, 16)] = zv

        @pl.loop(0, RPS // K2)
        def _(j):
            pltpu.sync_copy(msg, acc.at[pl.ds(row0 + j * K2, K2)])

        plsc.subcore_barrier()

        pltpu.async_copy(h4_hbm.at[p].at[srcall.at[0]], hrows0, semh0)
        pltpu.async_copy(_e_src(0), erow0.at[pl.ds(0, K2 * 8)], seme0)

        @pl.loop(0, NCHUNK2 // 2)
        def _(cc):
            for par in range(2):
                c = cc * 2 + par

                @pl.when(c + 1 < NCHUNK2)
                def _():
                    pltpu.async_copy(h4_hbm.at[p].at[srcall.at[c + 1]],
                                     hb[1 - par], sh[1 - par])
                    pltpu.async_copy(_e_src(c + 1),
                                     eb[1 - par].at[pl.ds(0, K2 * 8)],
                                     se[1 - par])

                pltpu.make_async_copy(h4_hbm.at[p].at[srcall.at[c]],
                                      hb[par], sh[par]).wait()
                pltpu.make_async_copy(_e_src(c),
                                      eb[par].at[pl.ds(0, K2 * 8)],
                                      se[par]).wait()

                @pl.loop(0, K2)
                def _(e):
                    ewv = eb[par][pl.ds(e * 8, 16)]
                    w0 = ewv[2 * p]
                    w1 = ewv[2 * p + 1]
                    for j in range(8):
                        hv = hb[par][e, pl.ds(j * 16, 16)]
                        msg[e, pl.ds(j * 16, 16)] = hv * (w0 if j < 4 else w1)

                pltpu.sync_copy(msg, acc.at[dstall.at[c]], add=True)

        plsc.subcore_barrier()
        pltpu.sync_copy(acc.at[pl.ds(row0, RPS)],
                        out_hbm.at[p, cid, pl.ds(row0, RPS)])


def kernel(x, edge_index, W1, att_src1, att_dst1, b1, W2, att_src2, att_dst2, b2):
    n = N
    loop = jnp.arange(n, dtype=jnp.int32)
    src = jnp.concatenate([edge_index[0].astype(jnp.int32), loop])
    dst = jnp.concatenate([edge_index[1].astype(jnp.int32), loop])
    src_p = jnp.concatenate([src, jnp.full((EP - NE,), n, jnp.int32)])
    dst_p = jnp.concatenate([dst, jnp.full((EP - NE,), n, jnp.int32)])
    src2d = src_p.reshape(32, NCHUNK, K)
    dst2d = dst_p.reshape(32, NCHUNK, K)
    src2b = src_p.reshape(32, NCHUNK2, K2)
    dst2b = dst_p.reshape(32, NCHUNK2, K2)

    A_src = jnp.zeros((HEADS * HID, HEADS), jnp.float32)
    A_src = A_src.at[jnp.arange(HEADS * HID), jnp.arange(HEADS * HID) // HID].set(
        att_src1.reshape(-1))
    A_dst = jnp.zeros((HEADS * HID, HEADS), jnp.float32)
    A_dst = A_dst.at[jnp.arange(HEADS * HID), jnp.arange(HEADS * HID) // HID].set(
        att_dst1.reshape(-1))
    Acat = jnp.concatenate([A_src, A_dst], axis=1)

    x_pad = jnp.pad(x, ((0, NP - n), (0, 0)))
    h4, asd = _node1(x_pad, W1, Acat)
    shift8 = jnp.max(asd[:n, :HEADS], axis=0) + jnp.max(asd[:n, HEADS:], axis=0)
    shift16 = jnp.concatenate([shift8, shift8])

    dsa = jnp.concatenate([asd[:, HEADS:], asd[:, :HEADS]], axis=1)

    e_flat, segp = _edge1(asd, dsa, src2d, dst2d, shift16)

    eps8 = 1e-16 * jnp.exp(-shift8)
    eps16 = jnp.concatenate([eps8, eps8])[None, :]
    SelAll = jnp.zeros((4, 16, 128), jnp.float32)
    for _p in range(4):
        SelAll = SelAll.at[_p, 2 * _p, 0:64].set(1.0)
        SelAll = SelAll.at[_p, 2 * _p + 1, 64:128].set(1.0)
    invex = _invex(segp, eps16, SelAll)

    out4, = _edge2(h4, e_flat, src2b, dst2b)

    W2ext = jnp.zeros((HEADS * HID, 128), jnp.float32)
    W2ext = W2ext.at[:, 0].set(W2[:, 0])
    W2ext = W2ext.at[:, 1].set(W2[:, 0] * att_src2[0, 0])
    W2ext = W2ext.at[:, 2].set(W2[:, 0] * att_dst2[0, 0])
    b1r = b1.reshape(4, 1, 128)
    h2a = _node2(out4, invex, b1r, W2ext)
    h2v = h2a[:, 0]
    a_src2 = h2a[:, 1]
    a_dst2 = h2a[:, 2]
    shift2 = jnp.max(a_src2[:n]) + jnp.max(a_dst2[:n])
    shift2v = jnp.full((16,), shift2, jnp.float32)

    e2_flat, seg2p = _edge3a(a_src2, a_dst2, src2d, dst2d, shift2v)
    eps2 = 1e-16 * jnp.exp(-shift2)
    eps2_128 = jnp.full((1, 128), eps2, jnp.float32)
    inv2full = _inv(seg2p, eps2_128)
    out2p, = _edge3b(h2v, e2_flat, src2d, dst2d)

    b2arr = jnp.full((1, 128), b2[0], jnp.float32)
    sig = _fin(out2p, inv2full, b2arr)
    return sig.reshape(NP, 16)[:n, 0]


# speedup vs baseline: 26.3443x; 1.2134x over previous
import functools

import jax
import jax.numpy as jnp
from jax import lax
from jax.experimental import pallas as pl
from jax.experimental.pallas import tpu as pltpu
from jax.experimental.pallas import tpu_sc as plsc

N = 10000
E = 320000
IN_CH = 128
HID = 64
HEADS = 8
OUT_CH = 1

NP = 10240
NE = E + N
EPT = 10496
EP = 32 * EPT
K = 128
NCHUNK = EPT // K
K2 = 64
NCHUNK2 = EPT // K2
RPS = NP // 16

ROWS = 512

_SC_PARAMS = pltpu.CompilerParams(
    use_tc_tiling_on_sc=False, needs_layout_passes=False)

_mesh = plsc.VectorSubcoreMesh(
    core_axis_name="c", subcore_axis_name="s", num_cores=2, num_subcores=16)


def _node1_body(x_ref, w_ref, acat_ref, h_ref, asd_ref):
    p = pl.program_id(1)
    hp = jnp.dot(x_ref[...], w_ref[...], preferred_element_type=jnp.float32)
    h_ref[0] = hp.astype(jnp.bfloat16)
    contrib = jnp.dot(hp, acat_ref[...], preferred_element_type=jnp.float32)

    @pl.when(p == 0)
    def _():
        asd_ref[...] = contrib

    @pl.when(p != 0)
    def _():
        asd_ref[...] += contrib


def _node1(x_pad, W1, Acat):
    return pl.pallas_call(
        _node1_body,
        grid=(NP // ROWS, 4),
        in_specs=[
            pl.BlockSpec((ROWS, IN_CH), lambda i, p: (i, 0)),
            pl.BlockSpec((IN_CH, 128), lambda i, p: (0, p)),
            pl.BlockSpec((128, 16), lambda i, p: (p, 0)),
        ],
        out_specs=[
            pl.BlockSpec((1, ROWS, 128), lambda i, p: (p, i, 0)),
            pl.BlockSpec((ROWS, 16), lambda i, p: (i, 0)),
        ],
        out_shape=[
            jax.ShapeDtypeStruct((4, NP, 128), jnp.bfloat16),
            jax.ShapeDtypeStruct((NP, 16), jnp.float32),
        ],
    )(x_pad, W1, Acat)


def _inv_body(segp_ref, eps_ref, inv_ref):
    inv_ref[...] = 1.0 / (segp_ref[0] + segp_ref[1] + eps_ref[...])


def _inv(segp, eps128):
    return pl.pallas_call(
        _inv_body,
        grid=(2,),
        in_specs=[
            pl.BlockSpec((2, NP * 16 // 256, 128), lambda i: (0, i, 0)),
            pl.BlockSpec((1, 128), lambda i: (0, 0)),
        ],
        out_specs=pl.BlockSpec((NP * 16 // 256, 128), lambda i: (i, 0)),
        out_shape=jax.ShapeDtypeStruct((NP * 16 // 128, 128), jnp.float32),
    )(segp.reshape(2, NP * 16 // 128, 128), eps128)


@functools.partial(
    pl.kernel,
    out_type=[
        jax.ShapeDtypeStruct((EP * 8,), jnp.float32),
        jax.ShapeDtypeStruct((2, NP, 16), jnp.float32),
    ],
    mesh=_mesh,
    compiler_params=_SC_PARAMS,
    scratch_types=[
        pltpu.VMEM((K,), jnp.int32),
        pltpu.VMEM((K,), jnp.int32),
        pltpu.VMEM((K, 16), jnp.float32),
        pltpu.VMEM((K, 16), jnp.float32),
        pltpu.VMEM((K, 16), jnp.float32),
        pltpu.VMEM((K * 8,), jnp.float32),
        pltpu.VMEM((64, 16), jnp.float32),
        pltpu.VMEM((16,), jnp.float32),
        pltpu.VMEM_SHARED((NP, 16), jnp.float32),
        pltpu.SemaphoreType.DMA,
    ],
)
def _edge1(asd_hbm, dsa_hbm, src_hbm, dst_hbm, shift_hbm,
           e_hbm, segp_hbm,
           srcv, dstv, srows, drows, ebuf, ecomp, zbuf, shiftbuf,
           seg_acc, sem):
    cid = lax.axis_index("c")
    sid = lax.axis_index("s")
    tile = cid * 16 + sid
    ebase = tile * EPT

    pltpu.sync_copy(shift_hbm, shiftbuf)

    zv = jnp.zeros((16,), jnp.float32)

    @pl.loop(0, 64)
    def _(r):
        zbuf[r, :] = zv

    row0 = sid * RPS

    @pl.loop(0, RPS // 64)
    def _(j):
        pltpu.sync_copy(zbuf, seg_acc.at[pl.ds(row0 + j * 64, 64)])

    @pl.loop(0, K)
    def _(r):
        ebuf[r, :] = zv

    plsc.subcore_barrier()

    shiftv = shiftbuf[...]
    lanes = lax.broadcasted_iota(jnp.int32, (16,), 0)
    headv = jnp.bitwise_and(lanes, 7)
    pat01 = lax.shift_right_logical(lanes, 3)

    @pl.loop(0, NCHUNK)
    def _(c):
        pltpu.sync_copy(src_hbm.at[tile, c], srcv)
        pltpu.sync_copy(dst_hbm.at[tile, c], dstv)
        pltpu.async_copy(asd_hbm.at[srcv], srows, sem).wait()
        pltpu.async_copy(dsa_hbm.at[dstv], drows, sem).wait()
        for i in range(K // 2):
            rowv = pat01 + (2 * i)
            sv = plsc.load_gather(srows, [rowv, headv])
            dv = plsc.load_gather(drows, [rowv, headv])
            al = sv + dv
            al = jnp.maximum(al, 0.2 * al)
            ev = jnp.exp(al - shiftv)
            plsc.store_scatter(ebuf, [rowv, headv], ev)
            ecomp[pl.ds(i * 16, 16)] = ev
        pltpu.sync_copy(ebuf, seg_acc.at[dstv], add=True)
        pltpu.sync_copy(ecomp, e_hbm.at[pl.ds((ebase + c * K) * 8, K * 8)])

    plsc.subcore_barrier()
    pltpu.sync_copy(seg_acc.at[pl.ds(row0, RPS)],
                    segp_hbm.at[cid, pl.ds(row0, RPS)])


def _node2_body(out4_ref, invex_ref, b1_ref, w2e_ref, h2a_ref):
    p = pl.program_id(1)
    blk = out4_ref[0]
    z = invex_ref[0] * (blk[0] + blk[1]) + b1_ref[0]
    z = jnp.where(z > 0, z, jnp.exp(jnp.minimum(z, 0.0)) - 1.0)
    contrib = jnp.dot(z, w2e_ref[...], preferred_element_type=jnp.float32)

    @pl.when(p == 0)
    def _():
        h2a_ref[...] = contrib

    @pl.when(p != 0)
    def _():
        h2a_ref[...] += contrib


def _node2(out4, invex, b1r, W2ext):
    return pl.pallas_call(
        _node2_body,
        grid=(NP // ROWS, 4),
        in_specs=[
            pl.BlockSpec((1, 2, ROWS, 128), lambda i, p: (p, 0, i, 0)),
            pl.BlockSpec((1, ROWS, 128), lambda i, p: (p, i, 0)),
            pl.BlockSpec((1, 1, 128), lambda i, p: (p, 0, 0)),
            pl.BlockSpec((128, 128), lambda i, p: (p, 0)),
        ],
        out_specs=pl.BlockSpec((ROWS, 128), lambda i, p: (i, 0)),
        out_shape=jax.ShapeDtypeStruct((NP, 128), jnp.float32),
    )(out4, invex, b1r, W2ext)


def _invex_body(segp_ref, eps_ref, sel_ref, out_ref):
    s = segp_ref[0] + segp_ref[1]
    iv = 1.0 / (s + eps_ref[...])
    out_ref[0] = jnp.dot(iv, sel_ref[0], preferred_element_type=jnp.float32)


def _invex(segp, eps16, SelAll):
    return pl.pallas_call(
        _invex_body,
        grid=(4, NP // 512),
        in_specs=[
            pl.BlockSpec((2, 512, 16), lambda p, i: (0, i, 0)),
            pl.BlockSpec((1, 16), lambda p, i: (0, 0)),
            pl.BlockSpec((1, 16, 128), lambda p, i: (p, 0, 0)),
        ],
        out_specs=pl.BlockSpec((1, 512, 128), lambda p, i: (p, i, 0)),
        out_shape=jax.ShapeDtypeStruct((4, NP, 128), jnp.float32),
    )(segp, eps16, SelAll)


def _fin_body(p_ref, inv_ref, b2_ref, o_ref):
    o_ref[...] = jax.nn.sigmoid(
        inv_ref[...] * (p_ref[0] + p_ref[1]) + b2_ref[...])


def _fin(out2p, inv2full, b2arr):
    return pl.pallas_call(
        _fin_body,
        grid=(2,),
        in_specs=[
            pl.BlockSpec((2, NP * 16 // 256, 128), lambda i: (0, i, 0)),
            pl.BlockSpec((NP * 16 // 256, 128), lambda i: (i, 0)),
            pl.BlockSpec((1, 128), lambda i: (0, 0)),
        ],
        out_specs=pl.BlockSpec((NP * 16 // 256, 128), lambda i: (i, 0)),
        out_shape=jax.ShapeDtypeStruct((NP * 16 // 128, 128), jnp.float32),
    )(out2p.reshape(2, NP * 16 // 128, 128), inv2full, b2arr)


@functools.partial(
    pl.kernel,
    out_type=[
        jax.ShapeDtypeStruct((EP,), jnp.float32),
        jax.ShapeDtypeStruct((2, NP, 16), jnp.float32),
    ],
    mesh=_mesh,
    compiler_params=_SC_PARAMS,
    scratch_types=[
        pltpu.VMEM((NP,), jnp.float32),
        pltpu.VMEM((NP,), jnp.float32),
        pltpu.VMEM((K,), jnp.int32),
        pltpu.VMEM((K,), jnp.int32),
        pltpu.VMEM((K, 16), jnp.float32),
        pltpu.VMEM((K,), jnp.float32),
        pltpu.VMEM((64, 16), jnp.float32),
        pltpu.VMEM((16,), jnp.float32),
        pltpu.VMEM_SHARED((NP, 16), jnp.float32),
    ],
)
def _edge3a(asrc2_hbm, adst2_hbm, src_hbm, dst_hbm, shift2_hbm,
            e2_hbm, seg2p_hbm,
            at, dt, srcv, dstv, ebuf, e2c, zbuf, shiftbuf, acc2):
    cid = lax.axis_index("c")
    sid = lax.axis_index("s")
    tile = cid * 16 + sid
    ebase = tile * EPT
    row0 = sid * RPS

    pltpu.sync_copy(asrc2_hbm, at)
    pltpu.sync_copy(adst2_hbm, dt)
    pltpu.sync_copy(shift2_hbm, shiftbuf)

    zv = jnp.zeros((16,), jnp.float32)

    @pl.loop(0, 64)
    def _(r):
        zbuf[r, :] = zv

    @pl.loop(0, RPS // 64)
    def _(j):
        pltpu.sync_copy(zbuf, acc2.at[pl.ds(row0 + j * 64, 64)])

    @pl.loop(0, K)
    def _(r):
        ebuf[r, :] = zv

    plsc.subcore_barrier()

    s2v = shiftbuf[...]
    lanes = lax.broadcasted_iota(jnp.int32, (16,), 0)
    zero16 = jnp.zeros((16,), jnp.int32)

    @pl.loop(0, NCHUNK)
    def _(c):
        pltpu.sync_copy(src_hbm.at[tile, c], srcv)
        pltpu.sync_copy(dst_hbm.at[tile, c], dstv)
        for i in range(K // 16):
            src16 = srcv[pl.ds(i * 16, 16)]
            dst16 = dstv[pl.ds(i * 16, 16)]
            sv = plsc.load_gather(at, [src16])
            dv = plsc.load_gather(dt, [dst16])
            al = sv + dv
            al = jnp.maximum(al, 0.2 * al)
            e2v = jnp.exp(al - s2v)
            e2c[pl.ds(i * 16, 16)] = e2v
            plsc.store_scatter(ebuf, [lanes + i * 16, zero16], e2v)
        pltpu.sync_copy(ebuf, acc2.at[dstv], add=True)
        pltpu.sync_copy(e2c, e2_hbm.at[pl.ds(ebase + c * K, K)])

    plsc.subcore_barrier()
    pltpu.sync_copy(acc2.at[pl.ds(row0, RPS)],
                    seg2p_hbm.at[cid, pl.ds(row0, RPS)])


@functools.partial(
    pl.kernel,
    out_type=[jax.ShapeDtypeStruct((2, NP, 16), jnp.float32)],
    mesh=_mesh,
    compiler_params=_SC_PARAMS,
    scratch_types=[
        pltpu.VMEM((NP,), jnp.float32),
        pltpu.VMEM((K,), jnp.int32),
        pltpu.VMEM((K,), jnp.int32),
        pltpu.VMEM((K,), jnp.float32),
        pltpu.VMEM((K, 16), jnp.float32),
        pltpu.VMEM((64, 16), jnp.float32),
        pltpu.VMEM_SHARED((NP, 16), jnp.float32),
    ],
)
def _edge3b(h2_hbm, e2_hbm, src_hbm, dst_hbm,
            out2p_hbm,
            h2t, srcv, dstv, e2c, mrow, zbuf, acc2):
    cid = lax.axis_index("c")
    sid = lax.axis_index("s")
    tile = cid * 16 + sid
    ebase = tile * EPT
    row0 = sid * RPS

    pltpu.sync_copy(h2_hbm, h2t)

    zv = jnp.zeros((16,), jnp.float32)

    @pl.loop(0, 64)
    def _(r):
        zbuf[r, :] = zv

    @pl.loop(0, RPS // 64)
    def _(j):
        pltpu.sync_copy(zbuf, acc2.at[pl.ds(row0 + j * 64, 64)])

    @pl.loop(0, K)
    def _(r):
        mrow[r, :] = zv

    plsc.subcore_barrier()

    lanes = lax.broadcasted_iota(jnp.int32, (16,), 0)
    zero16 = jnp.zeros((16,), jnp.int32)

    @pl.loop(0, NCHUNK)
    def _(c):
        pltpu.sync_copy(src_hbm.at[tile, c], srcv)
        pltpu.sync_copy(dst_hbm.at[tile, c], dstv)
        pltpu.sync_copy(e2_hbm.at[pl.ds(ebase + c * K, K)], e2c)
        for i in range(K // 16):
            src16 = srcv[pl.ds(i * 16, 16)]
            dst16 = dstv[pl.ds(i * 16, 16)]
            e2v = e2c[pl.ds(i * 16, 16)]
            m = e2v * plsc.load_gather(h2t, [src16])
            plsc.store_scatter(mrow, [lanes + i * 16, zero16], m)
        pltpu.sync_copy(mrow, acc2.at[dstv], add=True)

    plsc.subcore_barrier()
    pltpu.sync_copy(acc2.at[pl.ds(row0, RPS)],
                    out2p_hbm.at[cid, pl.ds(row0, RPS)])


@functools.partial(
    pl.kernel,
    out_type=[jax.ShapeDtypeStruct((4, 2, NP, 128), jnp.float32)],
    mesh=_mesh,
    compiler_params=_SC_PARAMS,
    scratch_types=[
        pltpu.VMEM((NCHUNK2, K2), jnp.int32),
        pltpu.VMEM((NCHUNK2, K2), jnp.int32),
        pltpu.VMEM((K2, 128), jnp.bfloat16),
        pltpu.VMEM((K2, 128), jnp.bfloat16),
        pltpu.VMEM((K2 * 8 + 16,), jnp.float32),
        pltpu.VMEM((K2 * 8 + 16,), jnp.float32),
        pltpu.VMEM((K2, 128), jnp.float32),
        pltpu.VMEM((K2, 128), jnp.float32),
        pltpu.VMEM_SHARED((NP, 128), jnp.float32),
        pltpu.SemaphoreType.DMA,
        pltpu.SemaphoreType.DMA,
        pltpu.SemaphoreType.DMA,
        pltpu.SemaphoreType.DMA,
        pltpu.SemaphoreType.DMA,
        pltpu.SemaphoreType.DMA,
    ],
)
def _edge2(h4_hbm, e_hbm, src_hbm, dst_hbm,
           out_hbm,
           srcall, dstall, hrows0, hrows1, erow0, erow1, msg0, msg1, acc,
           semh0, semh1, seme0, seme1, semm0, semm1):
    cid = lax.axis_index("c")
    sid = lax.axis_index("s")
    tile = cid * 16 + sid
    ebase = tile * EPT
    row0 = sid * RPS

    pltpu.sync_copy(src_hbm.at[tile], srcall)
    pltpu.sync_copy(dst_hbm.at[tile], dstall)

    hb = [hrows0, hrows1]
    eb = [erow0, erow1]
    mb = [msg0, msg1]
    sh = [semh0, semh1]
    se = [seme0, seme1]
    sm = [semm0, semm1]
    zv = jnp.zeros((16,), jnp.float32)
    maskhi = jnp.full((16,), -65536, jnp.int32)

    def _e_src(c):
        return e_hbm.at[pl.ds((ebase + c * K2) * 8, K2 * 8)]

    for p in range(4):
        @pl.loop(0, K2)
        def _(r):
            for j in range(8):
                msg0[r, pl.ds(j * 16, 16)] = zv

        @pl.loop(0, RPS // K2)
        def _(j):
            pltpu.sync_copy(msg0, acc.at[pl.ds(row0 + j * K2, K2)])

        plsc.subcore_barrier()

        pltpu.async_copy(h4_hbm.at[p].at[srcall.at[0]], hrows0, semh0)
        pltpu.async_copy(_e_src(0), erow0.at[pl.ds(0, K2 * 8)], seme0)

        @pl.loop(0, NCHUNK2 // 2)
        def _(cc):
            for par in range(2):
                c = cc * 2 + par

                @pl.when(c + 1 < NCHUNK2)
                def _():
                    pltpu.async_copy(h4_hbm.at[p].at[srcall.at[c + 1]],
                                     hb[1 - par], sh[1 - par])
                    pltpu.async_copy(_e_src(c + 1),
                                     eb[1 - par].at[pl.ds(0, K2 * 8)],
                                     se[1 - par])

                pltpu.make_async_copy(h4_hbm.at[p].at[srcall.at[c]],
                                      hb[par], sh[par]).wait()
                pltpu.make_async_copy(_e_src(c),
                                      eb[par].at[pl.ds(0, K2 * 8)],
                                      se[par]).wait()

                @pl.when(c >= 2)
                def _():
                    pltpu.make_async_copy(mb[par], acc.at[dstall.at[c]],
                                          sm[par]).wait()

                @pl.loop(0, K2)
                def _(e):
                    ewv = eb[par][pl.ds(e * 8, 16)]
                    w0 = ewv[2 * p]
                    w1 = ewv[2 * p + 1]
                    for g in range(4):
                        w = w0 if g < 2 else w1
                        hword = plsc.bitcast(
                            hb[par][e, pl.ds(g * 32, 32)], jnp.int32)
                        veven = plsc.bitcast(
                            lax.shift_left(hword, 16), jnp.float32) * w
                        vodd = plsc.bitcast(
                            jnp.bitwise_and(hword, maskhi), jnp.float32) * w
                        mb[par][e, pl.ds(g * 32, 16)] = veven
                        mb[par][e, pl.ds(g * 32 + 16, 16)] = vodd

                pltpu.async_copy(mb[par], acc.at[dstall.at[c]], sm[par],
                                 add=True)

        for par in range(2):
            pltpu.make_async_copy(mb[par], acc.at[dstall.at[0]],
                                  sm[par]).wait()

        plsc.subcore_barrier()
        pltpu.sync_copy(acc.at[pl.ds(row0, RPS)],
                        out_hbm.at[p, cid, pl.ds(row0, RPS)])


def kernel(x, edge_index, W1, att_src1, att_dst1, b1, W2, att_src2, att_dst2, b2):
    n = N
    loop = jnp.arange(n, dtype=jnp.int32)
    src = jnp.concatenate([edge_index[0].astype(jnp.int32), loop])
    dst = jnp.concatenate([edge_index[1].astype(jnp.int32), loop])
    src_p = jnp.concatenate([src, jnp.full((EP - NE,), n, jnp.int32)])
    dst_p = jnp.concatenate([dst, jnp.full((EP - NE,), n, jnp.int32)])
    src2d = src_p.reshape(32, NCHUNK, K)
    dst2d = dst_p.reshape(32, NCHUNK, K)
    src2b = src_p.reshape(32, NCHUNK2, K2)
    dst2b = dst_p.reshape(32, NCHUNK2, K2)

    A_src = jnp.zeros((HEADS * HID, HEADS), jnp.float32)
    A_src = A_src.at[jnp.arange(HEADS * HID), jnp.arange(HEADS * HID) // HID].set(
        att_src1.reshape(-1))
    A_dst = jnp.zeros((HEADS * HID, HEADS), jnp.float32)
    A_dst = A_dst.at[jnp.arange(HEADS * HID), jnp.arange(HEADS * HID) // HID].set(
        att_dst1.reshape(-1))
    Acat = jnp.concatenate([A_src, A_dst], axis=1)

    x_pad = jnp.pad(x, ((0, NP - n), (0, 0)))
    h4, asd = _node1(x_pad, W1, Acat)
    shift8 = jnp.max(asd[:n, :HEADS], axis=0) + jnp.max(asd[:n, HEADS:], axis=0)
    shift16 = jnp.concatenate([shift8, shift8])

    dsa = jnp.concatenate([asd[:, HEADS:], asd[:, :HEADS]], axis=1)

    e_flat, segp = _edge1(asd, dsa, src2d, dst2d, shift16)

    eps8 = 1e-16 * jnp.exp(-shift8)
    eps16 = jnp.concatenate([eps8, eps8])[None, :]
    SelAll = jnp.zeros((4, 16, 128), jnp.float32)
    for _p in range(4):
        SelAll = SelAll.at[_p, 2 * _p, 0:64].set(1.0)
        SelAll = SelAll.at[_p, 2 * _p + 1, 64:128].set(1.0)
    invex = _invex(segp, eps16, SelAll)

    out4, = _edge2(h4, e_flat, src2b, dst2b)

    W2ext = jnp.zeros((HEADS * HID, 128), jnp.float32)
    W2ext = W2ext.at[:, 0].set(W2[:, 0])
    W2ext = W2ext.at[:, 1].set(W2[:, 0] * att_src2[0, 0])
    W2ext = W2ext.at[:, 2].set(W2[:, 0] * att_dst2[0, 0])
    l = jnp.arange(128)
    g = l // 32
    k = l % 32
    chl = g * 32 + 2 * (k % 16) + (k // 16)
    perm512 = (jnp.arange(4)[:, None] * 128 + chl[None, :]).reshape(-1)
    W2p = W2ext[perm512, :]
    b1r = b1[perm512].reshape(4, 1, 128)
    h2a = _node2(out4, invex, b1r, W2p)
    h2v = h2a[:, 0]
    a_src2 = h2a[:, 1]
    a_dst2 = h2a[:, 2]
    shift2 = jnp.max(a_src2[:n]) + jnp.max(a_dst2[:n])
    shift2v = jnp.full((16,), shift2, jnp.float32)

    e2_flat, seg2p = _edge3a(a_src2, a_dst2, src2d, dst2d, shift2v)
    eps2 = 1e-16 * jnp.exp(-shift2)
    eps2_128 = jnp.full((1, 128), eps2, jnp.float32)
    inv2full = _inv(seg2p, eps2_128)
    out2p, = _edge3b(h2v, e2_flat, src2d, dst2d)

    b2arr = jnp.full((1, 128), b2[0], jnp.float32)
    sig = _fin(out2p, inv2full, b2arr)
    return sig.reshape(NP, 16)[:n, 0]


# speedup vs baseline: 31.7810x; 1.2064x over previous
import functools

import jax
import jax.numpy as jnp
from jax import lax
from jax.experimental import pallas as pl
from jax.experimental.pallas import tpu as pltpu
from jax.experimental.pallas import tpu_sc as plsc

N = 10000
E = 320000
IN_CH = 128
HID = 64
HEADS = 8
OUT_CH = 1

NP = 10240
NE = E + N
EPT = 10496
EP = 32 * EPT
K = 128
NCHUNK = EPT // K
K2 = 64
NCHUNK2 = EPT // K2
RPS = NP // 16

ROWS = 512

_SC_PARAMS = pltpu.CompilerParams(
    use_tc_tiling_on_sc=False, needs_layout_passes=False)

_mesh = plsc.VectorSubcoreMesh(
    core_axis_name="c", subcore_axis_name="s", num_cores=2, num_subcores=16)


def _node1_body(x_ref, w_ref, acat_ref, h_ref, asd_ref):
    p = pl.program_id(1)
    hp = jnp.dot(x_ref[...], w_ref[...], preferred_element_type=jnp.float32)
    h_ref[0] = hp.astype(jnp.bfloat16)
    contrib = jnp.dot(hp, acat_ref[...], preferred_element_type=jnp.float32)

    @pl.when(p == 0)
    def _():
        asd_ref[...] = contrib

    @pl.when(p != 0)
    def _():
        asd_ref[...] += contrib


def _node1(x_pad, W1, Acat):
    return pl.pallas_call(
        _node1_body,
        grid=(NP // ROWS, 4),
        in_specs=[
            pl.BlockSpec((ROWS, IN_CH), lambda i, p: (i, 0)),
            pl.BlockSpec((IN_CH, 128), lambda i, p: (0, p)),
            pl.BlockSpec((128, 16), lambda i, p: (p, 0)),
        ],
        out_specs=[
            pl.BlockSpec((1, ROWS, 128), lambda i, p: (p, i, 0)),
            pl.BlockSpec((ROWS, 16), lambda i, p: (i, 0)),
        ],
        out_shape=[
            jax.ShapeDtypeStruct((4, NP, 128), jnp.bfloat16),
            jax.ShapeDtypeStruct((NP, 16), jnp.float32),
        ],
    )(x_pad, W1, Acat)


def _inv_body(segp_ref, eps_ref, inv_ref):
    inv_ref[...] = 1.0 / (segp_ref[0] + segp_ref[1] + eps_ref[...])


def _inv(segp, eps128):
    return pl.pallas_call(
        _inv_body,
        grid=(2,),
        in_specs=[
            pl.BlockSpec((2, NP * 16 // 256, 128), lambda i: (0, i, 0)),
            pl.BlockSpec((1, 128), lambda i: (0, 0)),
        ],
        out_specs=pl.BlockSpec((NP * 16 // 256, 128), lambda i: (i, 0)),
        out_shape=jax.ShapeDtypeStruct((NP * 16 // 128, 128), jnp.float32),
    )(segp.reshape(2, NP * 16 // 128, 128), eps128)


@functools.partial(
    pl.kernel,
    out_type=[
        jax.ShapeDtypeStruct((EP * 8,), jnp.float32),
        jax.ShapeDtypeStruct((2, NP, 16), jnp.float32),
    ],
    mesh=_mesh,
    compiler_params=_SC_PARAMS,
    scratch_types=[
        pltpu.VMEM((NCHUNK, K), jnp.int32),
        pltpu.VMEM((NCHUNK, K), jnp.int32),
        pltpu.VMEM((K, 16), jnp.float32),
        pltpu.VMEM((K, 16), jnp.float32),
        pltpu.VMEM((K, 16), jnp.float32),
        pltpu.VMEM((K, 16), jnp.float32),
        pltpu.VMEM((K, 16), jnp.float32),
        pltpu.VMEM((K * 8,), jnp.float32),
        pltpu.VMEM((64, 16), jnp.float32),
        pltpu.VMEM((16,), jnp.float32),
        pltpu.VMEM_SHARED((NP, 16), jnp.float32),
        pltpu.SemaphoreType.DMA,
        pltpu.SemaphoreType.DMA,
        pltpu.SemaphoreType.DMA,
        pltpu.SemaphoreType.DMA,
    ],
)
def _edge1(asd_hbm, dsa_hbm, src_hbm, dst_hbm, shift_hbm,
           e_hbm, segp_hbm,
           srcall, dstall, srows0, srows1, drows0, drows1, ebuf, ecomp,
           zbuf, shiftbuf, seg_acc, sems0, sems1, semd0, semd1):
    cid = lax.axis_index("c")
    sid = lax.axis_index("s")
    tile = cid * 16 + sid
    ebase = tile * EPT

    pltpu.sync_copy(shift_hbm, shiftbuf)
    pltpu.sync_copy(src_hbm.at[tile], srcall)
    pltpu.sync_copy(dst_hbm.at[tile], dstall)

    sb = [srows0, srows1]
    db = [drows0, drows1]
    ss = [sems0, sems1]
    sd = [semd0, semd1]

    zv = jnp.zeros((16,), jnp.float32)

    @pl.loop(0, 64)
    def _(r):
        zbuf[r, :] = zv

    row0 = sid * RPS

    @pl.loop(0, RPS // 64)
    def _(j):
        pltpu.sync_copy(zbuf, seg_acc.at[pl.ds(row0 + j * 64, 64)])

    @pl.loop(0, K)
    def _(r):
        ebuf[r, :] = zv

    plsc.subcore_barrier()

    shiftv = shiftbuf[...]
    lanes = lax.broadcasted_iota(jnp.int32, (16,), 0)
    headv = jnp.bitwise_and(lanes, 7)
    pat01 = lax.shift_right_logical(lanes, 3)

    pltpu.async_copy(asd_hbm.at[srcall.at[0]], srows0, sems0)
    pltpu.async_copy(dsa_hbm.at[dstall.at[0]], drows0, semd0)

    @pl.loop(0, NCHUNK // 2)
    def _(cc):
        for par in range(2):
            c = cc * 2 + par

            @pl.when(c + 1 < NCHUNK)
            def _():
                pltpu.async_copy(asd_hbm.at[srcall.at[c + 1]],
                                 sb[1 - par], ss[1 - par])
                pltpu.async_copy(dsa_hbm.at[dstall.at[c + 1]],
                                 db[1 - par], sd[1 - par])

            pltpu.make_async_copy(asd_hbm.at[srcall.at[c]],
                                  sb[par], ss[par]).wait()
            pltpu.make_async_copy(dsa_hbm.at[dstall.at[c]],
                                  db[par], sd[par]).wait()

            for i in range(K // 2):
                rowv = pat01 + (2 * i)
                sv = plsc.load_gather(sb[par], [rowv, headv])
                dv = plsc.load_gather(db[par], [rowv, headv])
                al = sv + dv
                al = jnp.maximum(al, 0.2 * al)
                ev = jnp.exp(al - shiftv)
                plsc.store_scatter(ebuf, [rowv, headv], ev)
                ecomp[pl.ds(i * 16, 16)] = ev
            pltpu.sync_copy(ebuf, seg_acc.at[dstall.at[c]], add=True)
            pltpu.sync_copy(ecomp, e_hbm.at[pl.ds((ebase + c * K) * 8, K * 8)])

    plsc.subcore_barrier()
    pltpu.sync_copy(seg_acc.at[pl.ds(row0, RPS)],
                    segp_hbm.at[cid, pl.ds(row0, RPS)])


def _node2_body(out4_ref, invex_ref, b1_ref, w2e_ref, h2a_ref):
    p = pl.program_id(1)
    blk = out4_ref[0]
    z = invex_ref[0] * (blk[0] + blk[1]) + b1_ref[0]
    z = jnp.where(z > 0, z, jnp.exp(jnp.minimum(z, 0.0)) - 1.0)
    contrib = jnp.dot(z, w2e_ref[...], preferred_element_type=jnp.float32)

    @pl.when(p == 0)
    def _():
        h2a_ref[...] = contrib

    @pl.when(p != 0)
    def _():
        h2a_ref[...] += contrib


def _node2(out4, invex, b1r, W2ext):
    return pl.pallas_call(
        _node2_body,
        grid=(NP // ROWS, 4),
        in_specs=[
            pl.BlockSpec((1, 2, ROWS, 128), lambda i, p: (p, 0, i, 0)),
            pl.BlockSpec((1, ROWS, 128), lambda i, p: (p, i, 0)),
            pl.BlockSpec((1, 1, 128), lambda i, p: (p, 0, 0)),
            pl.BlockSpec((128, 128), lambda i, p: (p, 0)),
        ],
        out_specs=pl.BlockSpec((ROWS, 128), lambda i, p: (i, 0)),
        out_shape=jax.ShapeDtypeStruct((NP, 128), jnp.float32),
    )(out4, invex, b1r, W2ext)


def _invex_body(segp_ref, eps_ref, sel_ref, out_ref):
    s = segp_ref[0] + segp_ref[1]
    iv = 1.0 / (s + eps_ref[...])
    out_ref[0] = jnp.dot(iv, sel_ref[0], preferred_element_type=jnp.float32)


def _invex(segp, eps16, SelAll):
    return pl.pallas_call(
        _invex_body,
        grid=(4, NP // 512),
        in_specs=[
            pl.BlockSpec((2, 512, 16), lambda p, i: (0, i, 0)),
            pl.BlockSpec((1, 16), lambda p, i: (0, 0)),
            pl.BlockSpec((1, 16, 128), lambda p, i: (p, 0, 0)),
        ],
        out_specs=pl.BlockSpec((1, 512, 128), lambda p, i: (p, i, 0)),
        out_shape=jax.ShapeDtypeStruct((4, NP, 128), jnp.float32),
    )(segp, eps16, SelAll)


def _fin_body(p_ref, inv_ref, b2_ref, o_ref):
    o_ref[...] = jax.nn.sigmoid(
        inv_ref[...] * (p_ref[0] + p_ref[1]) + b2_ref[...])


def _fin(out2p, inv2full, b2arr):
    return pl.pallas_call(
        _fin_body,
        grid=(2,),
        in_specs=[
            pl.BlockSpec((2, NP * 16 // 256, 128), lambda i: (0, i, 0)),
            pl.BlockSpec((NP * 16 // 256, 128), lambda i: (i, 0)),
            pl.BlockSpec((1, 128), lambda i: (0, 0)),
        ],
        out_specs=pl.BlockSpec((NP * 16 // 256, 128), lambda i: (i, 0)),
        out_shape=jax.ShapeDtypeStruct((NP * 16 // 128, 128), jnp.float32),
    )(out2p.reshape(2, NP * 16 // 128, 128), inv2full, b2arr)


@functools.partial(
    pl.kernel,
    out_type=[
        jax.ShapeDtypeStruct((EP,), jnp.float32),
        jax.ShapeDtypeStruct((2, NP, 16), jnp.float32),
    ],
    mesh=_mesh,
    compiler_params=_SC_PARAMS,
    scratch_types=[
        pltpu.VMEM((NP,), jnp.float32),
        pltpu.VMEM((NP,), jnp.float32),
        pltpu.VMEM((NCHUNK, K), jnp.int32),
        pltpu.VMEM((NCHUNK, K), jnp.int32),
        pltpu.VMEM((K, 16), jnp.float32),
        pltpu.VMEM((K,), jnp.float32),
        pltpu.VMEM((64, 16), jnp.float32),
        pltpu.VMEM((16,), jnp.float32),
        pltpu.VMEM_SHARED((NP, 16), jnp.float32),
    ],
)
def _edge3a(asrc2_hbm, adst2_hbm, src_hbm, dst_hbm, shift2_hbm,
            e2_hbm, seg2p_hbm,
            at, dt, srcall, dstall, ebuf, e2c, zbuf, shiftbuf, acc2):
    cid = lax.axis_index("c")
    sid = lax.axis_index("s")
    tile = cid * 16 + sid
    ebase = tile * EPT
    row0 = sid * RPS

    pltpu.sync_copy(asrc2_hbm, at)
    pltpu.sync_copy(adst2_hbm, dt)
    pltpu.sync_copy(shift2_hbm, shiftbuf)
    pltpu.sync_copy(src_hbm.at[tile], srcall)
    pltpu.sync_copy(dst_hbm.at[tile], dstall)

    zv = jnp.zeros((16,), jnp.float32)

    @pl.loop(0, 64)
    def _(r):
        zbuf[r, :] = zv

    @pl.loop(0, RPS // 64)
    def _(j):
        pltpu.sync_copy(zbuf, acc2.at[pl.ds(row0 + j * 64, 64)])

    @pl.loop(0, K)
    def _(r):
        ebuf[r, :] = zv

    plsc.subcore_barrier()

    s2v = shiftbuf[...]
    lanes = lax.broadcasted_iota(jnp.int32, (16,), 0)
    zero16 = jnp.zeros((16,), jnp.int32)

    @pl.loop(0, NCHUNK)
    def _(c):
        for i in range(K // 16):
            src16 = srcall[c, pl.ds(i * 16, 16)]
            dst16 = dstall[c, pl.ds(i * 16, 16)]
            sv = plsc.load_gather(at, [src16])
            dv = plsc.load_gather(dt, [dst16])
            al = sv + dv
            al = jnp.maximum(al, 0.2 * al)
            e2v = jnp.exp(al - s2v)
            e2c[pl.ds(i * 16, 16)] = e2v
            plsc.store_scatter(ebuf, [lanes + i * 16, zero16], e2v)
        pltpu.sync_copy(ebuf, acc2.at[dstall.at[c]], add=True)
        pltpu.sync_copy(e2c, e2_hbm.at[pl.ds(ebase + c * K, K)])

    plsc.subcore_barrier()
    pltpu.sync_copy(acc2.at[pl.ds(row0, RPS)],
                    seg2p_hbm.at[cid, pl.ds(row0, RPS)])


@functools.partial(
    pl.kernel,
    out_type=[jax.ShapeDtypeStruct((2, NP, 16), jnp.float32)],
    mesh=_mesh,
    compiler_params=_SC_PARAMS,
    scratch_types=[
        pltpu.VMEM((NP,), jnp.float32),
        pltpu.VMEM((NCHUNK, K), jnp.int32),
        pltpu.VMEM((NCHUNK, K), jnp.int32),
        pltpu.VMEM((K,), jnp.float32),
        pltpu.VMEM((K, 16), jnp.float32),
        pltpu.VMEM((64, 16), jnp.float32),
        pltpu.VMEM_SHARED((NP, 16), jnp.float32),
    ],
)
def _edge3b(h2_hbm, e2_hbm, src_hbm, dst_hbm,
            out2p_hbm,
            h2t, srcall, dstall, e2c, mrow, zbuf, acc2):
    cid = lax.axis_index("c")
    sid = lax.axis_index("s")
    tile = cid * 16 + sid
    ebase = tile * EPT
    row0 = sid * RPS

    pltpu.sync_copy(h2_hbm, h2t)
    pltpu.sync_copy(src_hbm.at[tile], srcall)
    pltpu.sync_copy(dst_hbm.at[tile], dstall)

    zv = jnp.zeros((16,), jnp.float32)

    @pl.loop(0, 64)
    def _(r):
        zbuf[r, :] = zv

    @pl.loop(0, RPS // 64)
    def _(j):
        pltpu.sync_copy(zbuf, acc2.at[pl.ds(row0 + j * 64, 64)])

    @pl.loop(0, K)
    def _(r):
        mrow[r, :] = zv

    plsc.subcore_barrier()

    lanes = lax.broadcasted_iota(jnp.int32, (16,), 0)
    zero16 = jnp.zeros((16,), jnp.int32)

    @pl.loop(0, NCHUNK)
    def _(c):
        pltpu.sync_copy(e2_hbm.at[pl.ds(ebase + c * K, K)], e2c)
        for i in range(K // 16):
            src16 = srcall[c, pl.ds(i * 16, 16)]
            e2v = e2c[pl.ds(i * 16, 16)]
            m = e2v * plsc.load_gather(h2t, [src16])
            plsc.store_scatter(mrow, [lanes + i * 16, zero16], m)
        pltpu.sync_copy(mrow, acc2.at[dstall.at[c]], add=True)

    plsc.subcore_barrier()
    pltpu.sync_copy(acc2.at[pl.ds(row0, RPS)],
                    out2p_hbm.at[cid, pl.ds(row0, RPS)])


@functools.partial(
    pl.kernel,
    out_type=[jax.ShapeDtypeStruct((4, 2, NP, 128), jnp.float32)],
    mesh=_mesh,
    compiler_params=_SC_PARAMS,
    scratch_types=[
        pltpu.VMEM((NCHUNK2, K2), jnp.int32),
        pltpu.VMEM((NCHUNK2, K2), jnp.int32),
        pltpu.VMEM((K2, 128), jnp.bfloat16),
        pltpu.VMEM((K2, 128), jnp.bfloat16),
        pltpu.VMEM((K2 * 8 + 16,), jnp.float32),
        pltpu.VMEM((K2 * 8 + 16,), jnp.float32),
        pltpu.VMEM((K2, 128), jnp.float32),
        pltpu.VMEM((K2, 128), jnp.float32),
        pltpu.VMEM_SHARED((NP, 128), jnp.float32),
        pltpu.SemaphoreType.DMA,
        pltpu.SemaphoreType.DMA,
        pltpu.SemaphoreType.DMA,
        pltpu.SemaphoreType.DMA,
        pltpu.SemaphoreType.DMA,
        pltpu.SemaphoreType.DMA,
    ],
)
def _edge2(h4_hbm, e_hbm, src_hbm, dst_hbm,
           out_hbm,
           srcall, dstall, hrows0, hrows1, erow0, erow1, msg0, msg1, acc,
           semh0, semh1, seme0, seme1, semm0, semm1):
    cid = lax.axis_index("c")
    sid = lax.axis_index("s")
    tile = cid * 16 + sid
    ebase = tile * EPT
    row0 = sid * RPS

    pltpu.sync_copy(src_hbm.at[tile], srcall)
    pltpu.sync_copy(dst_hbm.at[tile], dstall)

    hb = [hrows0, hrows1]
    eb = [erow0, erow1]
    mb = [msg0, msg1]
    sh = [semh0, semh1]
    se = [seme0, seme1]
    sm = [semm0, semm1]
    zv = jnp.zeros((16,), jnp.float32)
    maskhi = jnp.full((16,), -65536, jnp.int32)

    def _e_src(c):
        return e_hbm.at[pl.ds((ebase + c * K2) * 8, K2 * 8)]

    for p in range(4):
        @pl.loop(0, K2)
        def _(r):
            for j in range(8):
                msg0[r, pl.ds(j * 16, 16)] = zv

        @pl.loop(0, RPS // K2)
        def _(j):
            pltpu.sync_copy(msg0, acc.at[pl.ds(row0 + j * K2, K2)])

        plsc.subcore_barrier()

        pltpu.async_copy(h4_hbm.at[p].at[srcall.at[0]], hrows0, semh0)
        pltpu.async_copy(_e_src(0), erow0.at[pl.ds(0, K2 * 8)], seme0)

        @pl.loop(0, NCHUNK2 // 2)
        def _(cc):
            for par in range(2):
                c = cc * 2 + par

                @pl.when(c + 1 < NCHUNK2)
                def _():
                    pltpu.async_copy(h4_hbm.at[p].at[srcall.at[c + 1]],
                                     hb[1 - par], sh[1 - par])
                    pltpu.async_copy(_e_src(c + 1),
                                     eb[1 - par].at[pl.ds(0, K2 * 8)],
                                     se[1 - par])

                pltpu.make_async_copy(h4_hbm.at[p].at[srcall.at[c]],
                                      hb[par], sh[par]).wait()
                pltpu.make_async_copy(_e_src(c),
                                      eb[par].at[pl.ds(0, K2 * 8)],
                                      se[par]).wait()

                @pl.when(c >= 2)
                def _():
                    pltpu.make_async_copy(mb[par], acc.at[dstall.at[c]],
                                          sm[par]).wait()

                @pl.loop(0, K2)
                def _(e):
                    ewv = eb[par][pl.ds(e * 8, 16)]
                    w0 = ewv[2 * p]
                    w1 = ewv[2 * p + 1]
                    for g in range(4):
                        w = w0 if g < 2 else w1
                        hword = plsc.bitcast(
                            hb[par][e, pl.ds(g * 32, 32)], jnp.int32)
                        veven = plsc.bitcast(
                            lax.shift_left(hword, 16), jnp.float32) * w
                        vodd = plsc.bitcast(
                            jnp.bitwise_and(hword, maskhi), jnp.float32) * w
                        mb[par][e, pl.ds(g * 32, 16)] = veven
                        mb[par][e, pl.ds(g * 32 + 16, 16)] = vodd

                pltpu.async_copy(mb[par], acc.at[dstall.at[c]], sm[par],
                                 add=True)

        for par in range(2):
            pltpu.make_async_copy(mb[par], acc.at[dstall.at[0]],
                                  sm[par]).wait()

        plsc.subcore_barrier()
        pltpu.sync_copy(acc.at[pl.ds(row0, RPS)],
                        out_hbm.at[p, cid, pl.ds(row0, RPS)])


def kernel(x, edge_index, W1, att_src1, att_dst1, b1, W2, att_src2, att_dst2, b2):
    n = N
    loop = jnp.arange(n, dtype=jnp.int32)
    src = jnp.concatenate([edge_index[0].astype(jnp.int32), loop])
    dst = jnp.concatenate([edge_index[1].astype(jnp.int32), loop])
    src_p = jnp.concatenate([src, jnp.full((EP - NE,), n, jnp.int32)])
    dst_p = jnp.concatenate([dst, jnp.full((EP - NE,), n, jnp.int32)])
    src2d = src_p.reshape(32, NCHUNK, K)
    dst2d = dst_p.reshape(32, NCHUNK, K)
    src2b = src_p.reshape(32, NCHUNK2, K2)
    dst2b = dst_p.reshape(32, NCHUNK2, K2)

    A_src = jnp.zeros((HEADS * HID, HEADS), jnp.float32)
    A_src = A_src.at[jnp.arange(HEADS * HID), jnp.arange(HEADS * HID) // HID].set(
        att_src1.reshape(-1))
    A_dst = jnp.zeros((HEADS * HID, HEADS), jnp.float32)
    A_dst = A_dst.at[jnp.arange(HEADS * HID), jnp.arange(HEADS * HID) // HID].set(
        att_dst1.reshape(-1))
    Acat = jnp.concatenate([A_src, A_dst], axis=1)

    x_pad = jnp.pad(x, ((0, NP - n), (0, 0)))
    h4, asd = _node1(x_pad, W1, Acat)
    shift8 = jnp.max(asd[:n, :HEADS], axis=0) + jnp.max(asd[:n, HEADS:], axis=0)
    shift16 = jnp.concatenate([shift8, shift8])

    dsa = jnp.concatenate([asd[:, HEADS:], asd[:, :HEADS]], axis=1)

    e_flat, segp = _edge1(asd, dsa, src2d, dst2d, shift16)

    eps8 = 1e-16 * jnp.exp(-shift8)
    eps16 = jnp.concatenate([eps8, eps8])[None, :]
    SelAll = jnp.zeros((4, 16, 128), jnp.float32)
    for _p in range(4):
        SelAll = SelAll.at[_p, 2 * _p, 0:64].set(1.0)
        SelAll = SelAll.at[_p, 2 * _p + 1, 64:128].set(1.0)
    invex = _invex(segp, eps16, SelAll)

    out4, = _edge2(h4, e_flat, src2b, dst2b)

    W2ext = jnp.zeros((HEADS * HID, 128), jnp.float32)
    W2ext = W2ext.at[:, 0].set(W2[:, 0])
    W2ext = W2ext.at[:, 1].set(W2[:, 0] * att_src2[0, 0])
    W2ext = W2ext.at[:, 2].set(W2[:, 0] * att_dst2[0, 0])
    l = jnp.arange(128)
    g = l // 32
    k = l % 32
    chl = g * 32 + 2 * (k % 16) + (k // 16)
    perm512 = (jnp.arange(4)[:, None] * 128 + chl[None, :]).reshape(-1)
    W2p = W2ext[perm512, :]
    b1r = b1[perm512].reshape(4, 1, 128)
    h2a = _node2(out4, invex, b1r, W2p)
    h2v = h2a[:, 0]
    a_src2 = h2a[:, 1]
    a_dst2 = h2a[:, 2]
    shift2 = jnp.max(a_src2[:n]) + jnp.max(a_dst2[:n])
    shift2v = jnp.full((16,), shift2, jnp.float32)

    e2_flat, seg2p = _edge3a(a_src2, a_dst2, src2d, dst2d, shift2v)
    eps2 = 1e-16 * jnp.exp(-shift2)
    eps2_128 = jnp.full((1, 128), eps2, jnp.float32)
    inv2full = _inv(seg2p, eps2_128)
    out2p, = _edge3b(h2v, e2_flat, src2d, dst2d)

    b2arr = jnp.full((1, 128), b2[0], jnp.float32)
    sig = _fin(out2p, inv2full, b2arr)
    return sig.reshape(NP, 16)[:n, 0]


# speedup vs baseline: 38.6949x; 1.2175x over previous
import functools

import jax
import jax.numpy as jnp
from jax import lax
from jax.experimental import pallas as pl
from jax.experimental.pallas import tpu as pltpu
from jax.experimental.pallas import tpu_sc as plsc

N = 10000
E = 320000
IN_CH = 128
HID = 64
HEADS = 8
OUT_CH = 1

NP = 10240
NE = E + N
EPT = 10496
EP = 32 * EPT
K = 128
NCHUNK = EPT // K
K2 = 64
NCHUNK2 = EPT // K2
RPS = NP // 16

ROWS = 512

_SC_PARAMS = pltpu.CompilerParams(
    use_tc_tiling_on_sc=False, needs_layout_passes=False)

_mesh = plsc.VectorSubcoreMesh(
    core_axis_name="c", subcore_axis_name="s", num_cores=2, num_subcores=16)


def _node1_body(x_ref, w_ref, acat_ref, h_ref, asd_ref):
    p = pl.program_id(1)
    hp = jnp.dot(x_ref[...], w_ref[...], preferred_element_type=jnp.float32)
    h_ref[0] = hp.astype(jnp.bfloat16)
    contrib = jnp.dot(hp, acat_ref[...], preferred_element_type=jnp.float32)

    @pl.when(p == 0)
    def _():
        asd_ref[...] = contrib

    @pl.when(p != 0)
    def _():
        asd_ref[...] += contrib


def _node1(x_pad, W1, Acat):
    return pl.pallas_call(
        _node1_body,
        grid=(NP // ROWS, 4),
        in_specs=[
            pl.BlockSpec((ROWS, IN_CH), lambda i, p: (i, 0)),
            pl.BlockSpec((IN_CH, 128), lambda i, p: (0, p)),
            pl.BlockSpec((128, 16), lambda i, p: (p, 0)),
        ],
        out_specs=[
            pl.BlockSpec((1, ROWS, 128), lambda i, p: (p, i, 0)),
            pl.BlockSpec((ROWS, 16), lambda i, p: (i, 0)),
        ],
        out_shape=[
            jax.ShapeDtypeStruct((4, NP, 128), jnp.bfloat16),
            jax.ShapeDtypeStruct((NP, 16), jnp.float32),
        ],
    )(x_pad, W1, Acat)


def _inv_body(segp_ref, eps_ref, inv_ref):
    inv_ref[...] = 1.0 / (segp_ref[0] + segp_ref[1] + eps_ref[...])


def _inv(segp, eps128):
    return pl.pallas_call(
        _inv_body,
        grid=(2,),
        in_specs=[
            pl.BlockSpec((2, NP * 16 // 256, 128), lambda i: (0, i, 0)),
            pl.BlockSpec((1, 128), lambda i: (0, 0)),
        ],
        out_specs=pl.BlockSpec((NP * 16 // 256, 128), lambda i: (i, 0)),
        out_shape=jax.ShapeDtypeStruct((NP * 16 // 128, 128), jnp.float32),
    )(segp.reshape(2, NP * 16 // 128, 128), eps128)


@functools.partial(
    pl.kernel,
    out_type=[
        jax.ShapeDtypeStruct((EP * 8,), jnp.float32),
        jax.ShapeDtypeStruct((2, NP, 16), jnp.float32),
    ],
    mesh=_mesh,
    compiler_params=_SC_PARAMS,
    scratch_types=[
        pltpu.VMEM((NCHUNK, K), jnp.int32),
        pltpu.VMEM((NCHUNK, K), jnp.int32),
        pltpu.VMEM((K, 16), jnp.float32),
        pltpu.VMEM((K, 16), jnp.float32),
        pltpu.VMEM((K, 16), jnp.float32),
        pltpu.VMEM((K, 16), jnp.float32),
        pltpu.VMEM((K, 16), jnp.float32),
        pltpu.VMEM((K * 8,), jnp.float32),
        pltpu.VMEM((64, 16), jnp.float32),
        pltpu.VMEM((16,), jnp.float32),
        pltpu.VMEM_SHARED((NP, 16), jnp.float32),
        pltpu.SemaphoreType.DMA,
        pltpu.SemaphoreType.DMA,
        pltpu.SemaphoreType.DMA,
        pltpu.SemaphoreType.DMA,
    ],
)
def _edge1(asd_hbm, dsa_hbm, src_hbm, dst_hbm, shift_hbm,
           e_hbm, segp_hbm,
           srcall, dstall, srows0, srows1, drows0, drows1, ebuf, ecomp,
           zbuf, shiftbuf, seg_acc, sems0, sems1, semd0, semd1):
    cid = lax.axis_index("c")
    sid = lax.axis_index("s")
    tile = cid * 16 + sid
    ebase = tile * EPT

    pltpu.sync_copy(shift_hbm, shiftbuf)
    pltpu.sync_copy(src_hbm.at[tile], srcall)
    pltpu.sync_copy(dst_hbm.at[tile], dstall)

    sb = [srows0, srows1]
    db = [drows0, drows1]
    ss = [sems0, sems1]
    sd = [semd0, semd1]

    zv = jnp.zeros((16,), jnp.float32)

    @pl.loop(0, 64)
    def _(r):
        zbuf[r, :] = zv

    row0 = sid * RPS

    @pl.loop(0, RPS // 64)
    def _(j):
        pltpu.sync_copy(zbuf, seg_acc.at[pl.ds(row0 + j * 64, 64)])

    @pl.loop(0, K)
    def _(r):
        ebuf[r, :] = zv

    plsc.subcore_barrier()

    shiftv = shiftbuf[...]
    lanes = lax.broadcasted_iota(jnp.int32, (16,), 0)
    headv = jnp.bitwise_and(lanes, 7)
    pat01 = lax.shift_right_logical(lanes, 3)

    pltpu.async_copy(asd_hbm.at[srcall.at[0]], srows0, sems0)
    pltpu.async_copy(dsa_hbm.at[dstall.at[0]], drows0, semd0)

    @pl.loop(0, NCHUNK // 2)
    def _(cc):
        for par in range(2):
            c = cc * 2 + par

            @pl.when(c + 1 < NCHUNK)
            def _():
                pltpu.async_copy(asd_hbm.at[srcall.at[c + 1]],
                                 sb[1 - par], ss[1 - par])
                pltpu.async_copy(dsa_hbm.at[dstall.at[c + 1]],
                                 db[1 - par], sd[1 - par])

            pltpu.make_async_copy(asd_hbm.at[srcall.at[c]],
                                  sb[par], ss[par]).wait()
            pltpu.make_async_copy(dsa_hbm.at[dstall.at[c]],
                                  db[par], sd[par]).wait()

            for i in range(K // 2):
                rowv = pat01 + (2 * i)
                sv = plsc.load_gather(sb[par], [rowv, headv])
                dv = plsc.load_gather(db[par], [rowv, headv])
                al = sv + dv
                al = jnp.maximum(al, 0.2 * al)
                ev = jnp.exp(al - shiftv)
                plsc.store_scatter(ebuf, [rowv, headv], ev)
                ecomp[pl.ds(i * 16, 16)] = ev
            pltpu.sync_copy(ebuf, seg_acc.at[dstall.at[c]], add=True)
            pltpu.sync_copy(ecomp, e_hbm.at[pl.ds((ebase + c * K) * 8, K * 8)])

    plsc.subcore_barrier()
    pltpu.sync_copy(seg_acc.at[pl.ds(row0, RPS)],
                    segp_hbm.at[cid, pl.ds(row0, RPS)])


def _node2_body(out4_ref, invex_ref, b1_ref, w2e_ref, h2a_ref):
    p = pl.program_id(1)
    blk = out4_ref[0]
    z = invex_ref[0] * (blk[0] + blk[1]) + b1_ref[0]
    z = jnp.where(z > 0, z, jnp.exp(jnp.minimum(z, 0.0)) - 1.0)
    contrib = jnp.dot(z, w2e_ref[...], preferred_element_type=jnp.float32)

    @pl.when(p == 0)
    def _():
        h2a_ref[...] = contrib

    @pl.when(p != 0)
    def _():
        h2a_ref[...] += contrib


def _node2(out4, invex, b1r, W2ext):
    return pl.pallas_call(
        _node2_body,
        grid=(NP // ROWS, 4),
        in_specs=[
            pl.BlockSpec((1, 2, ROWS, 128), lambda i, p: (p, 0, i, 0)),
            pl.BlockSpec((1, ROWS, 128), lambda i, p: (p, i, 0)),
            pl.BlockSpec((1, 1, 128), lambda i, p: (p, 0, 0)),
            pl.BlockSpec((128, 128), lambda i, p: (p, 0)),
        ],
        out_specs=pl.BlockSpec((ROWS, 128), lambda i, p: (i, 0)),
        out_shape=jax.ShapeDtypeStruct((NP, 128), jnp.float32),
    )(out4, invex, b1r, W2ext)


def _invex_body(segp_ref, eps_ref, sel_ref, out_ref):
    s = segp_ref[0] + segp_ref[1]
    iv = 1.0 / (s + eps_ref[...])
    out_ref[0] = jnp.dot(iv, sel_ref[0], preferred_element_type=jnp.float32)


def _invex(segp, eps16, SelAll):
    return pl.pallas_call(
        _invex_body,
        grid=(4, NP // 512),
        in_specs=[
            pl.BlockSpec((2, 512, 16), lambda p, i: (0, i, 0)),
            pl.BlockSpec((1, 16), lambda p, i: (0, 0)),
            pl.BlockSpec((1, 16, 128), lambda p, i: (p, 0, 0)),
        ],
        out_specs=pl.BlockSpec((1, 512, 128), lambda p, i: (p, i, 0)),
        out_shape=jax.ShapeDtypeStruct((4, NP, 128), jnp.float32),
    )(segp, eps16, SelAll)


def _fin_body(p_ref, inv_ref, b2_ref, o_ref):
    o_ref[...] = jax.nn.sigmoid(
        inv_ref[...] * (p_ref[0] + p_ref[1]) + b2_ref[...])


def _fin(out2p, inv2full, b2arr):
    return pl.pallas_call(
        _fin_body,
        grid=(2,),
        in_specs=[
            pl.BlockSpec((2, NP * 16 // 256, 128), lambda i: (0, i, 0)),
            pl.BlockSpec((NP * 16 // 256, 128), lambda i: (i, 0)),
            pl.BlockSpec((1, 128), lambda i: (0, 0)),
        ],
        out_specs=pl.BlockSpec((NP * 16 // 256, 128), lambda i: (i, 0)),
        out_shape=jax.ShapeDtypeStruct((NP * 16 // 128, 128), jnp.float32),
    )(out2p.reshape(2, NP * 16 // 128, 128), inv2full, b2arr)


@functools.partial(
    pl.kernel,
    out_type=[
        jax.ShapeDtypeStruct((EP,), jnp.float32),
        jax.ShapeDtypeStruct((2, NP, 16), jnp.float32),
    ],
    mesh=_mesh,
    compiler_params=_SC_PARAMS,
    scratch_types=[
        pltpu.VMEM((NP,), jnp.float32),
        pltpu.VMEM((NP,), jnp.float32),
        pltpu.VMEM((NCHUNK, K), jnp.int32),
        pltpu.VMEM((NCHUNK, K), jnp.int32),
        pltpu.VMEM((K, 16), jnp.float32),
        pltpu.VMEM((K,), jnp.float32),
        pltpu.VMEM((64, 16), jnp.float32),
        pltpu.VMEM((16,), jnp.float32),
        pltpu.VMEM_SHARED((NP, 16), jnp.float32),
    ],
)
def _edge3a(asrc2_hbm, adst2_hbm, src_hbm, dst_hbm, shift2_hbm,
            e2_hbm, seg2p_hbm,
            at, dt, srcall, dstall, ebuf, e2c, zbuf, shiftbuf, acc2):
    cid = lax.axis_index("c")
    sid = lax.axis_index("s")
    tile = cid * 16 + sid
    ebase = tile * EPT
    row0 = sid * RPS

    pltpu.sync_copy(asrc2_hbm, at)
    pltpu.sync_copy(adst2_hbm, dt)
    pltpu.sync_copy(shift2_hbm, shiftbuf)
    pltpu.sync_copy(src_hbm.at[tile], srcall)
    pltpu.sync_copy(dst_hbm.at[tile], dstall)

    zv = jnp.zeros((16,), jnp.float32)

    @pl.loop(0, 64)
    def _(r):
        zbuf[r, :] = zv

    @pl.loop(0, RPS // 64)
    def _(j):
        pltpu.sync_copy(zbuf, acc2.at[pl.ds(row0 + j * 64, 64)])

    @pl.loop(0, K)
    def _(r):
        ebuf[r, :] = zv

    plsc.subcore_barrier()

    s2v = shiftbuf[...]
    lanes = lax.broadcasted_iota(jnp.int32, (16,), 0)
    zero16 = jnp.zeros((16,), jnp.int32)

    @pl.loop(0, NCHUNK)
    def _(c):
        for i in range(K // 16):
            src16 = srcall[c, pl.ds(i * 16, 16)]
            dst16 = dstall[c, pl.ds(i * 16, 16)]
            sv = plsc.load_gather(at, [src16])
            dv = plsc.load_gather(dt, [dst16])
            al = sv + dv
            al = jnp.maximum(al, 0.2 * al)
            e2v = jnp.exp(al - s2v)
            e2c[pl.ds(i * 16, 16)] = e2v
            plsc.store_scatter(ebuf, [lanes + i * 16, zero16], e2v)
        pltpu.sync_copy(ebuf, acc2.at[dstall.at[c]], add=True)
        pltpu.sync_copy(e2c, e2_hbm.at[pl.ds(ebase + c * K, K)])

    plsc.subcore_barrier()
    pltpu.sync_copy(acc2.at[pl.ds(row0, RPS)],
                    seg2p_hbm.at[cid, pl.ds(row0, RPS)])


@functools.partial(
    pl.kernel,
    out_type=[jax.ShapeDtypeStruct((2, NP, 16), jnp.float32)],
    mesh=_mesh,
    compiler_params=_SC_PARAMS,
    scratch_types=[
        pltpu.VMEM((NP,), jnp.float32),
        pltpu.VMEM((NCHUNK, K), jnp.int32),
        pltpu.VMEM((NCHUNK, K), jnp.int32),
        pltpu.VMEM((K,), jnp.float32),
        pltpu.VMEM((K, 16), jnp.float32),
        pltpu.VMEM((64, 16), jnp.float32),
        pltpu.VMEM_SHARED((NP, 16), jnp.float32),
    ],
)
def _edge3b(h2_hbm, e2_hbm, src_hbm, dst_hbm,
            out2p_hbm,
            h2t, srcall, dstall, e2c, mrow, zbuf, acc2):
    cid = lax.axis_index("c")
    sid = lax.axis_index("s")
    tile = cid * 16 + sid
    ebase = tile * EPT
    row0 = sid * RPS

    pltpu.sync_copy(h2_hbm, h2t)
    pltpu.sync_copy(src_hbm.at[tile], srcall)
    pltpu.sync_copy(dst_hbm.at[tile], dstall)

    zv = jnp.zeros((16,), jnp.float32)

    @pl.loop(0, 64)
    def _(r):
        zbuf[r, :] = zv

    @pl.loop(0, RPS // 64)
    def _(j):
        pltpu.sync_copy(zbuf, acc2.at[pl.ds(row0 + j * 64, 64)])

    @pl.loop(0, K)
    def _(r):
        mrow[r, :] = zv

    plsc.subcore_barrier()

    lanes = lax.broadcasted_iota(jnp.int32, (16,), 0)
    zero16 = jnp.zeros((16,), jnp.int32)

    @pl.loop(0, NCHUNK)
    def _(c):
        pltpu.sync_copy(e2_hbm.at[pl.ds(ebase + c * K, K)], e2c)
        for i in range(K // 16):
            src16 = srcall[c, pl.ds(i * 16, 16)]
            e2v = e2c[pl.ds(i * 16, 16)]
            m = e2v * plsc.load_gather(h2t, [src16])
            plsc.store_scatter(mrow, [lanes + i * 16, zero16], m)
        pltpu.sync_copy(mrow, acc2.at[dstall.at[c]], add=True)

    plsc.subcore_barrier()
    pltpu.sync_copy(acc2.at[pl.ds(row0, RPS)],
                    out2p_hbm.at[cid, pl.ds(row0, RPS)])


@functools.partial(
    pl.kernel,
    out_type=[jax.ShapeDtypeStruct((4, 2, NP, 128), jnp.float32)],
    mesh=_mesh,
    compiler_params=_SC_PARAMS,
    scratch_types=[
        pltpu.VMEM((NCHUNK2, K2), jnp.int32),
        pltpu.VMEM((NCHUNK2, K2), jnp.int32),
        pltpu.VMEM((K2, 128), jnp.bfloat16),
        pltpu.VMEM((K2, 128), jnp.bfloat16),
        pltpu.VMEM((K2 * 8 + 16,), jnp.float32),
        pltpu.VMEM((K2 * 8 + 16,), jnp.float32),
        pltpu.VMEM((K2, 128), jnp.float32),
        pltpu.VMEM((K2, 128), jnp.float32),
        pltpu.VMEM_SHARED((NP, 128), jnp.float32),
        pltpu.SemaphoreType.DMA,
        pltpu.SemaphoreType.DMA,
        pltpu.SemaphoreType.DMA,
        pltpu.SemaphoreType.DMA,
        pltpu.SemaphoreType.DMA,
        pltpu.SemaphoreType.DMA,
    ],
)
def _edge2(h4_hbm, e_hbm, src_hbm, dst_hbm,
           out_hbm,
           srcall, dstall, hrows0, hrows1, erow0, erow1, msg0, msg1, acc,
           semh0, semh1, seme0, seme1, semm0, semm1):
    cid = lax.axis_index("c")
    sid = lax.axis_index("s")
    tile = cid * 16 + sid
    ebase = tile * EPT
    row0 = sid * RPS

    pltpu.sync_copy(src_hbm.at[tile], srcall)
    pltpu.sync_copy(dst_hbm.at[tile], dstall)

    hb = [hrows0, hrows1]
    eb = [erow0, erow1]
    mb = [msg0, msg1]
    sh = [semh0, semh1]
    se = [seme0, seme1]
    sm = [semm0, semm1]
    zv = jnp.zeros((16,), jnp.float32)
    maskhi = jnp.full((16,), -65536, jnp.int32)

    def _e_src(c):
        return e_hbm.at[pl.ds((ebase + c * K2) * 8, K2 * 8)]

    for p in range(4):
        @pl.loop(0, K2)
        def _(r):
            for j in range(8):
                msg0[r, pl.ds(j * 16, 16)] = zv

        @pl.loop(0, RPS // K2)
        def _(j):
            pltpu.sync_copy(msg0, acc.at[pl.ds(row0 + j * K2, K2)])

        plsc.subcore_barrier()

        pltpu.async_copy(h4_hbm.at[p].at[srcall.at[0]], hrows0, semh0)
        pltpu.async_copy(_e_src(0), erow0.at[pl.ds(0, K2 * 8)], seme0)

        @pl.loop(0, NCHUNK2 // 2)
        def _(cc):
            for par in range(2):
                c = cc * 2 + par

                @pl.when(c + 1 < NCHUNK2)
                def _():
                    pltpu.async_copy(h4_hbm.at[p].at[srcall.at[c + 1]],
                                     hb[1 - par], sh[1 - par])
                    pltpu.async_copy(_e_src(c + 1),
                                     eb[1 - par].at[pl.ds(0, K2 * 8)],
                                     se[1 - par])

                pltpu.make_async_copy(h4_hbm.at[p].at[srcall.at[c]],
                                      hb[par], sh[par]).wait()
                pltpu.make_async_copy(_e_src(c),
                                      eb[par].at[pl.ds(0, K2 * 8)],
                                      se[par]).wait()

                @pl.when(c >= 2)
                def _():
                    pltpu.make_async_copy(mb[par], acc.at[dstall.at[c]],
                                          sm[par]).wait()

                @plsc.parallel_loop(0, K2, unroll=2)
                def _(e):
                    ewv = eb[par][pl.ds(e * 8, 16)]
                    w0 = ewv[2 * p]
                    w1 = ewv[2 * p + 1]
                    for g in range(4):
                        w = w0 if g < 2 else w1
                        hword = plsc.bitcast(
                            hb[par][e, pl.ds(g * 32, 32)], jnp.int32)
                        veven = plsc.bitcast(
                            lax.shift_left(hword, 16), jnp.float32) * w
                        vodd = plsc.bitcast(
                            jnp.bitwise_and(hword, maskhi), jnp.float32) * w
                        mb[par][e, pl.ds(g * 32, 16)] = veven
                        mb[par][e, pl.ds(g * 32 + 16, 16)] = vodd

                pltpu.async_copy(mb[par], acc.at[dstall.at[c]], sm[par],
                                 add=True)

        for par in range(2):
            pltpu.make_async_copy(mb[par], acc.at[dstall.at[0]],
                                  sm[par]).wait()

        plsc.subcore_barrier()
        pltpu.sync_copy(acc.at[pl.ds(row0, RPS)],
                        out_hbm.at[p, cid, pl.ds(row0, RPS)])


def kernel(x, edge_index, W1, att_src1, att_dst1, b1, W2, att_src2, att_dst2, b2):
    n = N
    loop = jnp.arange(n, dtype=jnp.int32)
    src = jnp.concatenate([edge_index[0].astype(jnp.int32), loop])
    dst = jnp.concatenate([edge_index[1].astype(jnp.int32), loop])
    src_p = jnp.concatenate([src, jnp.full((EP - NE,), n, jnp.int32)])
    dst_p = jnp.concatenate([dst, jnp.full((EP - NE,), n, jnp.int32)])
    src2d = src_p.reshape(32, NCHUNK, K)
    dst2d = dst_p.reshape(32, NCHUNK, K)
    src2b = src_p.reshape(32, NCHUNK2, K2)
    dst2b = dst_p.reshape(32, NCHUNK2, K2)

    A_src = jnp.zeros((HEADS * HID, HEADS), jnp.float32)
    A_src = A_src.at[jnp.arange(HEADS * HID), jnp.arange(HEADS * HID) // HID].set(
        att_src1.reshape(-1))
    A_dst = jnp.zeros((HEADS * HID, HEADS), jnp.float32)
    A_dst = A_dst.at[jnp.arange(HEADS * HID), jnp.arange(HEADS * HID) // HID].set(
        att_dst1.reshape(-1))
    Acat = jnp.concatenate([A_src, A_dst], axis=1)

    x_pad = jnp.pad(x, ((0, NP - n), (0, 0)))
    h4, asd = _node1(x_pad, W1, Acat)
    shift8 = jnp.max(asd[:n, :HEADS], axis=0) + jnp.max(asd[:n, HEADS:], axis=0)
    shift16 = jnp.concatenate([shift8, shift8])

    dsa = jnp.concatenate([asd[:, HEADS:], asd[:, :HEADS]], axis=1)

    e_flat, segp = _edge1(asd, dsa, src2d, dst2d, shift16)

    eps8 = 1e-16 * jnp.exp(-shift8)
    eps16 = jnp.concatenate([eps8, eps8])[None, :]
    SelAll = jnp.zeros((4, 16, 128), jnp.float32)
    for _p in range(4):
        SelAll = SelAll.at[_p, 2 * _p, 0:64].set(1.0)
        SelAll = SelAll.at[_p, 2 * _p + 1, 64:128].set(1.0)
    invex = _invex(segp, eps16, SelAll)

    out4, = _edge2(h4, e_flat, src2b, dst2b)

    W2ext = jnp.zeros((HEADS * HID, 128), jnp.float32)
    W2ext = W2ext.at[:, 0].set(W2[:, 0])
    W2ext = W2ext.at[:, 1].set(W2[:, 0] * att_src2[0, 0])
    W2ext = W2ext.at[:, 2].set(W2[:, 0] * att_dst2[0, 0])
    l = jnp.arange(128)
    g = l // 32
    k = l % 32
    chl = g * 32 + 2 * (k % 16) + (k // 16)
    perm512 = (jnp.arange(4)[:, None] * 128 + chl[None, :]).reshape(-1)
    W2p = W2ext[perm512, :]
    b1r = b1[perm512].reshape(4, 1, 128)
    h2a = _node2(out4, invex, b1r, W2p)
    h2v = h2a[:, 0]
    a_src2 = h2a[:, 1]
    a_dst2 = h2a[:, 2]
    shift2 = jnp.max(a_src2[:n]) + jnp.max(a_dst2[:n])
    shift2v = jnp.full((16,), shift2, jnp.float32)

    e2_flat, seg2p = _edge3a(a_src2, a_dst2, src2d, dst2d, shift2v)
    eps2 = 1e-16 * jnp.exp(-shift2)
    eps2_128 = jnp.full((1, 128), eps2, jnp.float32)
    inv2full = _inv(seg2p, eps2_128)
    out2p, = _edge3b(h2v, e2_flat, src2d, dst2d)

    b2arr = jnp.full((1, 128), b2[0], jnp.float32)
    sig = _fin(out2p, inv2full, b2arr)
    return sig.reshape(NP, 16)[:n, 0]


# speedup vs baseline: 40.4755x; 1.0460x over previous
import functools

import jax
import jax.numpy as jnp
from jax import lax
from jax.experimental import pallas as pl
from jax.experimental.pallas import tpu as pltpu
from jax.experimental.pallas import tpu_sc as plsc

N = 10000
E = 320000
IN_CH = 128
HID = 64
HEADS = 8
OUT_CH = 1

NP = 10240
NE = E + N
EPT = 10496
EP = 32 * EPT
K = 128
NCHUNK = EPT // K
K2 = 64
NCHUNK2 = EPT // K2
RPS = NP // 16

ROWS = 512

_SC_PARAMS = pltpu.CompilerParams(
    use_tc_tiling_on_sc=False, needs_layout_passes=False)

_mesh = plsc.VectorSubcoreMesh(
    core_axis_name="c", subcore_axis_name="s", num_cores=2, num_subcores=16)


def _node1_body(x_ref, w_ref, acat_ref, h_ref, asd_ref):
    p = pl.program_id(1)
    hp = jnp.dot(x_ref[...], w_ref[...], preferred_element_type=jnp.float32)
    h_ref[0] = hp.astype(jnp.bfloat16)
    contrib = jnp.dot(hp, acat_ref[...], preferred_element_type=jnp.float32)

    @pl.when(p == 0)
    def _():
        asd_ref[...] = contrib

    @pl.when(p != 0)
    def _():
        asd_ref[...] += contrib


def _node1(x_pad, W1, Acat):
    return pl.pallas_call(
        _node1_body,
        grid=(NP // ROWS, 4),
        in_specs=[
            pl.BlockSpec((ROWS, IN_CH), lambda i, p: (i, 0)),
            pl.BlockSpec((IN_CH, 128), lambda i, p: (0, p)),
            pl.BlockSpec((128, 16), lambda i, p: (p, 0)),
        ],
        out_specs=[
            pl.BlockSpec((1, ROWS, 128), lambda i, p: (p, i, 0)),
            pl.BlockSpec((ROWS, 16), lambda i, p: (i, 0)),
        ],
        out_shape=[
            jax.ShapeDtypeStruct((4, NP, 128), jnp.bfloat16),
            jax.ShapeDtypeStruct((NP, 16), jnp.float32),
        ],
    )(x_pad, W1, Acat)


def _inv_body(segp_ref, eps_ref, inv_ref):
    inv_ref[...] = 1.0 / (segp_ref[0] + segp_ref[1] + eps_ref[...])


def _inv(segp, eps128):
    return pl.pallas_call(
        _inv_body,
        grid=(2,),
        in_specs=[
            pl.BlockSpec((2, NP * 16 // 256, 128), lambda i: (0, i, 0)),
            pl.BlockSpec((1, 128), lambda i: (0, 0)),
        ],
        out_specs=pl.BlockSpec((NP * 16 // 256, 128), lambda i: (i, 0)),
        out_shape=jax.ShapeDtypeStruct((NP * 16 // 128, 128), jnp.float32),
    )(segp.reshape(2, NP * 16 // 128, 128), eps128)


@functools.partial(
    pl.kernel,
    out_type=[
        jax.ShapeDtypeStruct((EP * 8,), jnp.float32),
        jax.ShapeDtypeStruct((2, NP, 16), jnp.float32),
    ],
    mesh=_mesh,
    compiler_params=_SC_PARAMS,
    scratch_types=[
        pltpu.VMEM((NCHUNK, K), jnp.int32),
        pltpu.VMEM((NCHUNK, K), jnp.int32),
        pltpu.VMEM((K, 16), jnp.float32),
        pltpu.VMEM((K, 16), jnp.float32),
        pltpu.VMEM((K, 16), jnp.float32),
        pltpu.VMEM((K, 16), jnp.float32),
        pltpu.VMEM((K, 16), jnp.float32),
        pltpu.VMEM((K * 8,), jnp.float32),
        pltpu.VMEM((64, 16), jnp.float32),
        pltpu.VMEM((16,), jnp.float32),
        pltpu.VMEM_SHARED((NP, 16), jnp.float32),
        pltpu.SemaphoreType.DMA,
        pltpu.SemaphoreType.DMA,
        pltpu.SemaphoreType.DMA,
        pltpu.SemaphoreType.DMA,
    ],
)
def _edge1(asd_hbm, dsa_hbm, src_hbm, dst_hbm, shift_hbm,
           e_hbm, segp_hbm,
           srcall, dstall, srows0, srows1, drows0, drows1, ebuf, ecomp,
           zbuf, shiftbuf, seg_acc, sems0, sems1, semd0, semd1):
    cid = lax.axis_index("c")
    sid = lax.axis_index("s")
    tile = cid * 16 + sid
    ebase = tile * EPT

    pltpu.sync_copy(shift_hbm, shiftbuf)
    pltpu.sync_copy(src_hbm.at[tile], srcall)
    pltpu.sync_copy(dst_hbm.at[tile], dstall)

    sb = [srows0, srows1]
    db = [drows0, drows1]
    ss = [sems0, sems1]
    sd = [semd0, semd1]

    zv = jnp.zeros((16,), jnp.float32)

    @pl.loop(0, 64)
    def _(r):
        zbuf[r, :] = zv

    row0 = sid * RPS

    @pl.loop(0, RPS // 64)
    def _(j):
        pltpu.sync_copy(zbuf, seg_acc.at[pl.ds(row0 + j * 64, 64)])

    @pl.loop(0, K)
    def _(r):
        ebuf[r, :] = zv

    plsc.subcore_barrier()

    shiftv = shiftbuf[...]
    lanes = lax.broadcasted_iota(jnp.int32, (16,), 0)
    headv = jnp.bitwise_and(lanes, 7)
    pat01 = lax.shift_right_logical(lanes, 3)

    pltpu.async_copy(asd_hbm.at[srcall.at[0]], srows0, sems0)
    pltpu.async_copy(dsa_hbm.at[dstall.at[0]], drows0, semd0)

    @pl.loop(0, NCHUNK // 2)
    def _(cc):
        for par in range(2):
            c = cc * 2 + par

            @pl.when(c + 1 < NCHUNK)
            def _():
                pltpu.async_copy(asd_hbm.at[srcall.at[c + 1]],
                                 sb[1 - par], ss[1 - par])
                pltpu.async_copy(dsa_hbm.at[dstall.at[c + 1]],
                                 db[1 - par], sd[1 - par])

            pltpu.make_async_copy(asd_hbm.at[srcall.at[c]],
                                  sb[par], ss[par]).wait()
            pltpu.make_async_copy(dsa_hbm.at[dstall.at[c]],
                                  db[par], sd[par]).wait()

            @plsc.parallel_loop(0, K // 2, unroll=2)
            def _(i):
                rowv = pat01 + (2 * i)
                sv = plsc.load_gather(sb[par], [rowv, headv])
                dv = plsc.load_gather(db[par], [rowv, headv])
                al = sv + dv
                al = jnp.maximum(al, 0.2 * al)
                ev = jnp.exp(al - shiftv)
                plsc.store_scatter(ebuf, [rowv, headv], ev)
                ecomp[pl.ds(i * 16, 16)] = ev
            pltpu.sync_copy(ebuf, seg_acc.at[dstall.at[c]], add=True)
            pltpu.sync_copy(ecomp, e_hbm.at[pl.ds((ebase + c * K) * 8, K * 8)])

    plsc.subcore_barrier()
    pltpu.sync_copy(seg_acc.at[pl.ds(row0, RPS)],
                    segp_hbm.at[cid, pl.ds(row0, RPS)])


def _node2_body(out4_ref, invex_ref, b1_ref, w2e_ref, h2a_ref):
    p = pl.program_id(1)
    blk = out4_ref[0]
    z = invex_ref[0] * (blk[0] + blk[1]) + b1_ref[0]
    z = jnp.where(z > 0, z, jnp.exp(jnp.minimum(z, 0.0)) - 1.0)
    contrib = jnp.dot(z, w2e_ref[...], preferred_element_type=jnp.float32)

    @pl.when(p == 0)
    def _():
        h2a_ref[...] = contrib

    @pl.when(p != 0)
    def _():
        h2a_ref[...] += contrib


def _node2(out4, invex, b1r, W2ext):
    return pl.pallas_call(
        _node2_body,
        grid=(NP // ROWS, 4),
        in_specs=[
            pl.BlockSpec((1, 2, ROWS, 128), lambda i, p: (p, 0, i, 0)),
            pl.BlockSpec((1, ROWS, 128), lambda i, p: (p, i, 0)),
            pl.BlockSpec((1, 1, 128), lambda i, p: (p, 0, 0)),
            pl.BlockSpec((128, 128), lambda i, p: (p, 0)),
        ],
        out_specs=pl.BlockSpec((ROWS, 128), lambda i, p: (i, 0)),
        out_shape=jax.ShapeDtypeStruct((NP, 128), jnp.float32),
    )(out4, invex, b1r, W2ext)


def _invex_body(segp_ref, eps_ref, sel_ref, out_ref):
    s = segp_ref[0] + segp_ref[1]
    iv = 1.0 / (s + eps_ref[...])
    out_ref[0] = jnp.dot(iv, sel_ref[0], preferred_element_type=jnp.float32)


def _invex(segp, eps16, SelAll):
    return pl.pallas_call(
        _invex_body,
        grid=(4, NP // 512),
        in_specs=[
            pl.BlockSpec((2, 512, 16), lambda p, i: (0, i, 0)),
            pl.BlockSpec((1, 16), lambda p, i: (0, 0)),
            pl.BlockSpec((1, 16, 128), lambda p, i: (p, 0, 0)),
        ],
        out_specs=pl.BlockSpec((1, 512, 128), lambda p, i: (p, i, 0)),
        out_shape=jax.ShapeDtypeStruct((4, NP, 128), jnp.float32),
    )(segp, eps16, SelAll)


def _fin_body(p_ref, inv_ref, b2_ref, o_ref):
    o_ref[...] = jax.nn.sigmoid(
        inv_ref[...] * (p_ref[0] + p_ref[1]) + b2_ref[...])


def _fin(out2p, inv2full, b2arr):
    return pl.pallas_call(
        _fin_body,
        grid=(2,),
        in_specs=[
            pl.BlockSpec((2, NP * 16 // 256, 128), lambda i: (0, i, 0)),
            pl.BlockSpec((NP * 16 // 256, 128), lambda i: (i, 0)),
            pl.BlockSpec((1, 128), lambda i: (0, 0)),
        ],
        out_specs=pl.BlockSpec((NP * 16 // 256, 128), lambda i: (i, 0)),
        out_shape=jax.ShapeDtypeStruct((NP * 16 // 128, 128), jnp.float32),
    )(out2p.reshape(2, NP * 16 // 128, 128), inv2full, b2arr)


@functools.partial(
    pl.kernel,
    out_type=[
        jax.ShapeDtypeStruct((EP,), jnp.float32),
        jax.ShapeDtypeStruct((2, NP, 16), jnp.float32),
    ],
    mesh=_mesh,
    compiler_params=_SC_PARAMS,
    scratch_types=[
        pltpu.VMEM((NP,), jnp.float32),
        pltpu.VMEM((NP,), jnp.float32),
        pltpu.VMEM((NCHUNK, K), jnp.int32),
        pltpu.VMEM((NCHUNK, K), jnp.int32),
        pltpu.VMEM((K, 16), jnp.float32),
        pltpu.VMEM((K,), jnp.float32),
        pltpu.VMEM((64, 16), jnp.float32),
        pltpu.VMEM((16,), jnp.float32),
        pltpu.VMEM_SHARED((NP, 16), jnp.float32),
    ],
)
def _edge3a(asrc2_hbm, adst2_hbm, src_hbm, dst_hbm, shift2_hbm,
            e2_hbm, seg2p_hbm,
            at, dt, srcall, dstall, ebuf, e2c, zbuf, shiftbuf, acc2):
    cid = lax.axis_index("c")
    sid = lax.axis_index("s")
    tile = cid * 16 + sid
    ebase = tile * EPT
    row0 = sid * RPS

    pltpu.sync_copy(asrc2_hbm, at)
    pltpu.sync_copy(adst2_hbm, dt)
    pltpu.sync_copy(shift2_hbm, shiftbuf)
    pltpu.sync_copy(src_hbm.at[tile], srcall)
    pltpu.sync_copy(dst_hbm.at[tile], dstall)

    zv = jnp.zeros((16,), jnp.float32)

    @pl.loop(0, 64)
    def _(r):
        zbuf[r, :] = zv

    @pl.loop(0, RPS // 64)
    def _(j):
        pltpu.sync_copy(zbuf, acc2.at[pl.ds(row0 + j * 64, 64)])

    @pl.loop(0, K)
    def _(r):
        ebuf[r, :] = zv

    plsc.subcore_barrier()

    s2v = shiftbuf[...]
    lanes = lax.broadcasted_iota(jnp.int32, (16,), 0)
    zero16 = jnp.zeros((16,), jnp.int32)

    @pl.loop(0, NCHUNK)
    def _(c):
        @plsc.parallel_loop(0, K // 16, unroll=2)
        def _(i):
            src16 = srcall[c, pl.ds(i * 16, 16)]
            dst16 = dstall[c, pl.ds(i * 16, 16)]
            sv = plsc.load_gather(at, [src16])
            dv = plsc.load_gather(dt, [dst16])
            al = sv + dv
            al = jnp.maximum(al, 0.2 * al)
            e2v = jnp.exp(al - s2v)
            e2c[pl.ds(i * 16, 16)] = e2v
            plsc.store_scatter(ebuf, [lanes + i * 16, zero16], e2v)
        pltpu.sync_copy(ebuf, acc2.at[dstall.at[c]], add=True)
        pltpu.sync_copy(e2c, e2_hbm.at[pl.ds(ebase + c * K, K)])

    plsc.subcore_barrier()
    pltpu.sync_copy(acc2.at[pl.ds(row0, RPS)],
                    seg2p_hbm.at[cid, pl.ds(row0, RPS)])


@functools.partial(
    pl.kernel,
    out_type=[jax.ShapeDtypeStruct((2, NP, 16), jnp.float32)],
    mesh=_mesh,
    compiler_params=_SC_PARAMS,
    scratch_types=[
        pltpu.VMEM((NP,), jnp.float32),
        pltpu.VMEM((NCHUNK, K), jnp.int32),
        pltpu.VMEM((NCHUNK, K), jnp.int32),
        pltpu.VMEM((K,), jnp.float32),
        pltpu.VMEM((K, 16), jnp.float32),
        pltpu.VMEM((64, 16), jnp.float32),
        pltpu.VMEM_SHARED((NP, 16), jnp.float32),
    ],
)
def _edge3b(h2_hbm, e2_hbm, src_hbm, dst_hbm,
            out2p_hbm,
            h2t, srcall, dstall, e2c, mrow, zbuf, acc2):
    cid = lax.axis_index("c")
    sid = lax.axis_index("s")
    tile = cid * 16 + sid
    ebase = tile * EPT
    row0 = sid * RPS

    pltpu.sync_copy(h2_hbm, h2t)
    pltpu.sync_copy(src_hbm.at[tile], srcall)
    pltpu.sync_copy(dst_hbm.at[tile], dstall)

    zv = jnp.zeros((16,), jnp.float32)

    @pl.loop(0, 64)
    def _(r):
        zbuf[r, :] = zv

    @pl.loop(0, RPS // 64)
    def _(j):
        pltpu.sync_copy(zbuf, acc2.at[pl.ds(row0 + j * 64, 64)])

    @pl.loop(0, K)
    def _(r):
        mrow[r, :] = zv

    plsc.subcore_barrier()

    lanes = lax.broadcasted_iota(jnp.int32, (16,), 0)
    zero16 = jnp.zeros((16,), jnp.int32)

    @pl.loop(0, NCHUNK)
    def _(c):
        pltpu.sync_copy(e2_hbm.at[pl.ds(ebase + c * K, K)], e2c)
        @plsc.parallel_loop(0, K // 16, unroll=2)
        def _(i):
            src16 = srcall[c, pl.ds(i * 16, 16)]
            e2v = e2c[pl.ds(i * 16, 16)]
            m = e2v * plsc.load_gather(h2t, [src16])
            plsc.store_scatter(mrow, [lanes + i * 16, zero16], m)
        pltpu.sync_copy(mrow, acc2.at[dstall.at[c]], add=True)

    plsc.subcore_barrier()
    pltpu.sync_copy(acc2.at[pl.ds(row0, RPS)],
                    out2p_hbm.at[cid, pl.ds(row0, RPS)])


@functools.partial(
    pl.kernel,
    out_type=[jax.ShapeDtypeStruct((4, 2, NP, 128), jnp.float32)],
    mesh=_mesh,
    compiler_params=_SC_PARAMS,
    scratch_types=[
        pltpu.VMEM((NCHUNK2, K2), jnp.int32),
        pltpu.VMEM((NCHUNK2, K2), jnp.int32),
        pltpu.VMEM((K2, 128), jnp.bfloat16),
        pltpu.VMEM((K2, 128), jnp.bfloat16),
        pltpu.VMEM((K2 * 8 + 16,), jnp.float32),
        pltpu.VMEM((K2 * 8 + 16,), jnp.float32),
        pltpu.VMEM((K2, 128), jnp.float32),
        pltpu.VMEM((K2, 128), jnp.float32),
        pltpu.VMEM_SHARED((NP, 128), jnp.float32),
        pltpu.SemaphoreType.DMA,
        pltpu.SemaphoreType.DMA,
        pltpu.SemaphoreType.DMA,
        pltpu.SemaphoreType.DMA,
        pltpu.SemaphoreType.DMA,
        pltpu.SemaphoreType.DMA,
    ],
)
def _edge2(h4_hbm, e_hbm, src_hbm, dst_hbm,
           out_hbm,
           srcall, dstall, hrows0, hrows1, erow0, erow1, msg0, msg1, acc,
           semh0, semh1, seme0, seme1, semm0, semm1):
    cid = lax.axis_index("c")
    sid = lax.axis_index("s")
    tile = cid * 16 + sid
    ebase = tile * EPT
    row0 = sid * RPS

    pltpu.sync_copy(src_hbm.at[tile], srcall)
    pltpu.sync_copy(dst_hbm.at[tile], dstall)

    hb = [hrows0, hrows1]
    eb = [erow0, erow1]
    mb = [msg0, msg1]
    sh = [semh0, semh1]
    se = [seme0, seme1]
    sm = [semm0, semm1]
    zv = jnp.zeros((16,), jnp.float32)
    maskhi = jnp.full((16,), -65536, jnp.int32)

    def _e_src(c):
        return e_hbm.at[pl.ds((ebase + c * K2) * 8, K2 * 8)]

    for p in range(4):
        @pl.loop(0, K2)
        def _(r):
            for j in range(8):
                msg0[r, pl.ds(j * 16, 16)] = zv

        @pl.loop(0, RPS // K2)
        def _(j):
            pltpu.sync_copy(msg0, acc.at[pl.ds(row0 + j * K2, K2)])

        plsc.subcore_barrier()

        pltpu.async_copy(h4_hbm.at[p].at[srcall.at[0]], hrows0, semh0)
        pltpu.async_copy(_e_src(0), erow0.at[pl.ds(0, K2 * 8)], seme0)

        @pl.loop(0, NCHUNK2 // 2)
        def _(cc):
            for par in range(2):
                c = cc * 2 + par

                @pl.when(c + 1 < NCHUNK2)
                def _():
                    pltpu.async_copy(h4_hbm.at[p].at[srcall.at[c + 1]],
                                     hb[1 - par], sh[1 - par])
                    pltpu.async_copy(_e_src(c + 1),
                                     eb[1 - par].at[pl.ds(0, K2 * 8)],
                                     se[1 - par])

                pltpu.make_async_copy(h4_hbm.at[p].at[srcall.at[c]],
                                      hb[par], sh[par]).wait()
                pltpu.make_async_copy(_e_src(c),
                                      eb[par].at[pl.ds(0, K2 * 8)],
                                      se[par]).wait()

                @pl.when(c >= 2)
                def _():
                    pltpu.make_async_copy(mb[par], acc.at[dstall.at[c]],
                                          sm[par]).wait()

                @plsc.parallel_loop(0, K2, unroll=2)
                def _(e):
                    ewv = eb[par][pl.ds(e * 8, 16)]
                    w0 = ewv[2 * p]
                    w1 = ewv[2 * p + 1]
                    for g in range(4):
                        w = w0 if g < 2 else w1
                        hword = plsc.bitcast(
                            hb[par][e, pl.ds(g * 32, 32)], jnp.int32)
                        veven = plsc.bitcast(
                            lax.shift_left(hword, 16), jnp.float32) * w
                        vodd = plsc.bitcast(
                            jnp.bitwise_and(hword, maskhi), jnp.float32) * w
                        mb[par][e, pl.ds(g * 32, 16)] = veven
                        mb[par][e, pl.ds(g * 32 + 16, 16)] = vodd

                pltpu.async_copy(mb[par], acc.at[dstall.at[c]], sm[par],
                                 add=True)

        for par in range(2):
            pltpu.make_async_copy(mb[par], acc.at[dstall.at[0]],
                                  sm[par]).wait()

        plsc.subcore_barrier()
        pltpu.sync_copy(acc.at[pl.ds(row0, RPS)],
                        out_hbm.at[p, cid, pl.ds(row0, RPS)])


def kernel(x, edge_index, W1, att_src1, att_dst1, b1, W2, att_src2, att_dst2, b2):
    n = N
    loop = jnp.arange(n, dtype=jnp.int32)
    src = jnp.concatenate([edge_index[0].astype(jnp.int32), loop])
    dst = jnp.concatenate([edge_index[1].astype(jnp.int32), loop])
    src_p = jnp.concatenate([src, jnp.full((EP - NE,), n, jnp.int32)])
    dst_p = jnp.concatenate([dst, jnp.full((EP - NE,), n, jnp.int32)])
    src2d = src_p.reshape(32, NCHUNK, K)
    dst2d = dst_p.reshape(32, NCHUNK, K)
    src2b = src_p.reshape(32, NCHUNK2, K2)
    dst2b = dst_p.reshape(32, NCHUNK2, K2)

    A_src = jnp.zeros((HEADS * HID, HEADS), jnp.float32)
    A_src = A_src.at[jnp.arange(HEADS * HID), jnp.arange(HEADS * HID) // HID].set(
        att_src1.reshape(-1))
    A_dst = jnp.zeros((HEADS * HID, HEADS), jnp.float32)
    A_dst = A_dst.at[jnp.arange(HEADS * HID), jnp.arange(HEADS * HID) // HID].set(
        att_dst1.reshape(-1))
    Acat = jnp.concatenate([A_src, A_dst], axis=1)

    x_pad = jnp.pad(x, ((0, NP - n), (0, 0)))
    h4, asd = _node1(x_pad, W1, Acat)
    shift8 = jnp.max(asd[:n, :HEADS], axis=0) + jnp.max(asd[:n, HEADS:], axis=0)
    shift16 = jnp.concatenate([shift8, shift8])

    dsa = jnp.concatenate([asd[:, HEADS:], asd[:, :HEADS]], axis=1)

    e_flat, segp = _edge1(asd, dsa, src2d, dst2d, shift16)

    eps8 = 1e-16 * jnp.exp(-shift8)
    eps16 = jnp.concatenate([eps8, eps8])[None, :]
    SelAll = jnp.zeros((4, 16, 128), jnp.float32)
    for _p in range(4):
        SelAll = SelAll.at[_p, 2 * _p, 0:64].set(1.0)
        SelAll = SelAll.at[_p, 2 * _p + 1, 64:128].set(1.0)
    invex = _invex(segp, eps16, SelAll)

    out4, = _edge2(h4, e_flat, src2b, dst2b)

    W2ext = jnp.zeros((HEADS * HID, 128), jnp.float32)
    W2ext = W2ext.at[:, 0].set(W2[:, 0])
    W2ext = W2ext.at[:, 1].set(W2[:, 0] * att_src2[0, 0])
    W2ext = W2ext.at[:, 2].set(W2[:, 0] * att_dst2[0, 0])
    l = jnp.arange(128)
    g = l // 32
    k = l % 32
    chl = g * 32 + 2 * (k % 16) + (k // 16)
    perm512 = (jnp.arange(4)[:, None] * 128 + chl[None, :]).reshape(-1)
    W2p = W2ext[perm512, :]
    b1r = b1[perm512].reshape(4, 1, 128)
    h2a = _node2(out4, invex, b1r, W2p)
    h2v = h2a[:, 0]
    a_src2 = h2a[:, 1]
    a_dst2 = h2a[:, 2]
    shift2 = jnp.max(a_src2[:n]) + jnp.max(a_dst2[:n])
    shift2v = jnp.full((16,), shift2, jnp.float32)

    e2_flat, seg2p = _edge3a(a_src2, a_dst2, src2d, dst2d, shift2v)
    eps2 = 1e-16 * jnp.exp(-shift2)
    eps2_128 = jnp.full((1, 128), eps2, jnp.float32)
    inv2full = _inv(seg2p, eps2_128)
    out2p, = _edge3b(h2v, e2_flat, src2d, dst2d)

    b2arr = jnp.full((1, 128), b2[0], jnp.float32)
    sig = _fin(out2p, inv2full, b2arr)
    return sig.reshape(NP, 16)[:n, 0]
